# Initial kernel scaffold; baseline (speedup 1.0000x reference)
import jax
import jax.numpy as jnp
from jax.experimental import pallas as pl


def kernel(x, edge_index, W1, b1, Wg, att_src, att_dst, bg, Wl, Wr, bs):
    raise NotImplementedError("write your pallas kernel here")


# speedup vs baseline: 30.4673x; 30.4673x over previous
import functools

import jax
import jax.numpy as jnp
from jax import lax
from jax.experimental import pallas as pl
from jax.experimental.pallas import tpu as pltpu
from jax.experimental.pallas import tpu_sc as plsc

N = 10000
E = 320000
IN = 128
HID = 8
HEADS = 8
F = 64
ROWW = 128

NC = 2
NS = 16
NW = NC * NS
CH = 128
CHUNKS = -(-E // (NW * CH))
E_PAD = NW * CHUNKS * CH
RPT = 8 * (-(-(N + 1) // (NS * 8)))
NROWS = RPT * NS
BN = 400
GN = N // BN

_mesh = plsc.VectorSubcoreMesh(
    core_axis_name="c", subcore_axis_name="s", num_cores=NC, num_subcores=NS)
_sc_params = pltpu.CompilerParams(use_tc_tiling_on_sc=False)


@functools.partial(
    pl.kernel,
    out_type=jax.ShapeDtypeStruct((NC, NROWS, ROWW), jnp.float32),
    mesh=_mesh,
    compiler_params=_sc_params,
    scratch_types=[
        pltpu.VMEM((CH,), jnp.int32),
        pltpu.VMEM((CH,), jnp.int32),
        pltpu.VMEM((CH, F), jnp.float32),
        pltpu.VMEM((CH, F), jnp.float32),
        pltpu.VMEM((CH, F), jnp.float32),
        pltpu.VMEM((CH, ROWW), jnp.float32),
        pltpu.VMEM_SHARED((NROWS, ROWW), jnp.float32),
        pltpu.SemaphoreType.DMA,
    ],
)
def _sc1(src_hbm, dst_hbm, asx_hbm, adxp_hbm, h_hbm, zeros_hbm, out_hbm,
         src_v, dst_v, as_v, ad_v, h_v, row_v, tbl, sem):
    cid = lax.axis_index("c")
    sid = lax.axis_index("s")
    wid = cid * NS + sid
    r0 = pl.multiple_of(sid * RPT, 8)
    pltpu.sync_copy(zeros_hbm.at[pl.ds(r0, RPT)], tbl.at[pl.ds(r0, RPT)])

    plsc.subcore_barrier()

    def chunk(i, carry):
        base = pl.multiple_of((wid * CHUNKS + i) * CH, CH)
        pltpu.sync_copy(src_hbm.at[pl.ds(base, CH)], src_v)
        pltpu.sync_copy(dst_hbm.at[pl.ds(base, CH)], dst_v)
        c1 = pltpu.async_copy(asx_hbm.at[src_v], as_v, sem)
        c2 = pltpu.async_copy(adxp_hbm.at[dst_v], ad_v, sem)
        c3 = pltpu.async_copy(h_hbm.at[src_v], h_v, sem)
        c1.wait()
        c2.wait()
        c3.wait()

        def edge(r, carry2):
            for k in range(F // 16):
                a = as_v[r, pl.ds(16 * k, 16)] + ad_v[r, pl.ds(16 * k, 16)]
                s = jnp.exp(jnp.maximum(a, 0.2 * a))
                row_v[r, pl.ds(F + 16 * k, 16)] = s
                row_v[r, pl.ds(16 * k, 16)] = s * h_v[r, pl.ds(16 * k, 16)]
            return carry2

        lax.fori_loop(0, CH, edge, 0)
        pltpu.sync_copy(row_v, tbl.at[dst_v], add=True)
        return carry

    lax.fori_loop(0, CHUNKS, chunk, 0)
    plsc.subcore_barrier()
    pltpu.sync_copy(tbl.at[pl.ds(r0, RPT)], out_hbm.at[cid, pl.ds(r0, RPT)])


@functools.partial(
    pl.kernel,
    out_type=(jax.ShapeDtypeStruct((NC, NROWS, F), jnp.float32),
              jax.ShapeDtypeStruct((NC, NROWS, 16), jnp.float32)),
    mesh=_mesh,
    compiler_params=_sc_params,
    scratch_types=[
        pltpu.VMEM((CH,), jnp.int32),
        pltpu.VMEM((CH,), jnp.int32),
        pltpu.VMEM((CH, F), jnp.float32),
        pltpu.VMEM((CH, 16), jnp.float32),
        pltpu.VMEM_SHARED((NROWS, F), jnp.float32),
        pltpu.VMEM_SHARED((NROWS, 16), jnp.float32),
        pltpu.SemaphoreType.DMA,
    ],
)
def _sc2(src_hbm, dst_hbm, x32_hbm, zeros_hbm, z16_hbm, out_hbm, deg_hbm,
         src_v, dst_v, g_v, ones_v, tbl, dtbl, sem):
    cid = lax.axis_index("c")
    sid = lax.axis_index("s")
    wid = cid * NS + sid
    r0 = pl.multiple_of(sid * RPT, 8)
    pltpu.sync_copy(zeros_hbm.at[pl.ds(r0, RPT)], tbl.at[pl.ds(r0, RPT)])
    pltpu.sync_copy(z16_hbm.at[pl.ds(r0, RPT)], dtbl.at[pl.ds(r0, RPT)])

    one_hot = jnp.where(lax.iota(jnp.int32, 16) == 0, 1.0, 0.0)

    def preset(r, carry):
        ones_v[r, pl.ds(0, 16)] = one_hot
        return carry

    lax.fori_loop(0, CH, preset, 0)
    plsc.subcore_barrier()

    def chunk(i, carry):
        base = pl.multiple_of((wid * CHUNKS + i) * CH, CH)
        pltpu.sync_copy(src_hbm.at[pl.ds(base, CH)], src_v)
        pltpu.sync_copy(dst_hbm.at[pl.ds(base, CH)], dst_v)
        pltpu.async_copy(x32_hbm.at[src_v], g_v, sem).wait()
        pltpu.sync_copy(g_v, tbl.at[dst_v], add=True)
        pltpu.sync_copy(ones_v, dtbl.at[dst_v], add=True)
        return carry

    lax.fori_loop(0, CHUNKS, chunk, 0)
    plsc.subcore_barrier()
    pltpu.sync_copy(tbl.at[pl.ds(r0, RPT)], out_hbm.at[cid, pl.ds(r0, RPT)])
    pltpu.sync_copy(dtbl.at[pl.ds(r0, RPT)], deg_hbm.at[cid, pl.ds(r0, RPT)])


def _tca_body(x_ref, wg_ref, ws_ref, wd_ref,
              h_ref, asx_ref, adx_ref, sxsl_ref, usl_ref):
    xb = x_ref[...]
    h = jnp.dot(xb, wg_ref[...], preferred_element_type=jnp.float32)
    asx = jnp.dot(xb, ws_ref[...], preferred_element_type=jnp.float32)
    adx = jnp.dot(xb, wd_ref[...], preferred_element_type=jnp.float32)
    e = asx + adx
    sx = jnp.exp(jnp.maximum(e, 0.2 * e))
    h_ref[...] = h
    asx_ref[...] = asx
    adx_ref[...] = adx
    sxsl_ref[...] = sx
    usl_ref[...] = h * sx


def _tcb_body(p_ref, sxsl_ref, usl_ref, bg_ref, x32_ref):
    u = p_ref[0, :, 0:F] + p_ref[1, :, 0:F] + usl_ref[...]
    den = p_ref[0, :, F:2 * F] + p_ref[1, :, F:2 * F] + sxsl_ref[...]
    x32_ref[...] = jnp.maximum(u / den + bg_ref[...], 0.0)


def _tcc_body(aggp_ref, degp_ref, x32_ref, wlt_ref, wrt_ref, bs_ref, out_ref):
    deg = degp_ref[0, :, 0:1] + degp_ref[1, :, 0:1]
    inv = jnp.broadcast_to(1.0 / jnp.maximum(deg, 1.0), (BN, F))
    mean = (aggp_ref[0] + aggp_ref[1]) * inv
    out_ref[...] = (
        jnp.dot(mean, wlt_ref[...], preferred_element_type=jnp.float32)
        + jnp.dot(x32_ref[...], wrt_ref[...], preferred_element_type=jnp.float32)
        + bs_ref[...])


def kernel(x, edge_index, W1, b1, Wg, att_src, att_dst, bg, Wl, Wr, bs):
    eye8 = jnp.eye(HEADS, dtype=jnp.float32)
    ones8 = jnp.ones((1, 1, 1, HID), dtype=jnp.float32)
    m_s = (att_src[:, :, None, None] * eye8[:, None, :, None] * ones8
           ).reshape(F, F)
    m_d = (att_dst[:, :, None, None] * eye8[:, None, :, None] * ones8
           ).reshape(F, F)
    ws_x = Wg @ m_s
    wd_x = Wg @ m_d

    src = jnp.concatenate(
        [edge_index[0], jnp.zeros((E_PAD - E,), edge_index.dtype)])
    dst = jnp.concatenate(
        [edge_index[1], jnp.full((E_PAD - E,), N, edge_index.dtype)])
    src = src.astype(jnp.int32)
    dst = dst.astype(jnp.int32)
    z128 = jnp.zeros((NROWS, ROWW), jnp.float32)
    z64 = jnp.zeros((NROWS, F), jnp.float32)
    z16 = jnp.zeros((NROWS, 16), jnp.float32)

    h, asx, adx, sx_sl, u_sl = pl.pallas_call(
        _tca_body,
        grid=(GN,),
        in_specs=[
            pl.BlockSpec((BN, IN), lambda i: (i, 0)),
            pl.BlockSpec((IN, F), lambda i: (0, 0)),
            pl.BlockSpec((IN, F), lambda i: (0, 0)),
            pl.BlockSpec((IN, F), lambda i: (0, 0)),
        ],
        out_specs=[pl.BlockSpec((BN, F), lambda i: (i, 0))] * 5,
        out_shape=[jax.ShapeDtypeStruct((N, F), jnp.float32)] * 5,
    )(x, Wg, ws_x, wd_x)

    adx_p = jnp.concatenate([adx, jnp.zeros((1, F), jnp.float32)])

    part1 = _sc1(src, dst, asx, adx_p, h, z128)

    x32 = pl.pallas_call(
        _tcb_body,
        grid=(GN,),
        in_specs=[
            pl.BlockSpec((NC, BN, ROWW), lambda i: (0, i, 0)),
            pl.BlockSpec((BN, F), lambda i: (i, 0)),
            pl.BlockSpec((BN, F), lambda i: (i, 0)),
            pl.BlockSpec((1, F), lambda i: (0, 0)),
        ],
        out_specs=pl.BlockSpec((BN, F), lambda i: (i, 0)),
        out_shape=jax.ShapeDtypeStruct((N, F), jnp.float32),
    )(part1, sx_sl, u_sl, bg.reshape(1, F))

    part2, degp = _sc2(src, dst, x32, z64, z16)

    out = pl.pallas_call(
        _tcc_body,
        grid=(GN,),
        in_specs=[
            pl.BlockSpec((NC, BN, F), lambda i: (0, i, 0)),
            pl.BlockSpec((NC, BN, 16), lambda i: (0, i, 0)),
            pl.BlockSpec((BN, F), lambda i: (i, 0)),
            pl.BlockSpec((F, F), lambda i: (0, 0)),
            pl.BlockSpec((F, F), lambda i: (0, 0)),
            pl.BlockSpec((1, F), lambda i: (0, 0)),
        ],
        out_specs=pl.BlockSpec((BN, F), lambda i: (i, 0)),
        out_shape=jax.ShapeDtypeStruct((N, F), jnp.float32),
    )(part2, degp, x32, Wl.T, Wr.T, bs.reshape(1, F))

    return (x32, out)


# speedup vs baseline: 37.7723x; 1.2398x over previous
import functools

import jax
import jax.numpy as jnp
from jax import lax
from jax.experimental import pallas as pl
from jax.experimental.pallas import tpu as pltpu
from jax.experimental.pallas import tpu_sc as plsc

N = 10000
E = 320000
IN = 128
HID = 8
HEADS = 8
F = 64
ROWW = 128

NC = 2
NS = 16
NW = NC * NS
CH = 80
CHUNKS = 128
E_PAD = NW * CHUNKS * CH
RPT = 8 * (-(-(N + 1) // (NS * 8)))
NROWS = RPT * NS
BN = 400
GN = N // BN

_mesh = plsc.VectorSubcoreMesh(
    core_axis_name="c", subcore_axis_name="s", num_cores=NC, num_subcores=NS)
_sc_params = pltpu.CompilerParams(use_tc_tiling_on_sc=False)


@functools.partial(
    pl.kernel,
    out_type=jax.ShapeDtypeStruct((NC, NROWS, ROWW), jnp.float32),
    mesh=_mesh,
    compiler_params=_sc_params,
    scratch_types=[
        pltpu.VMEM((CHUNKS, CH), jnp.int32),
        pltpu.VMEM((CHUNKS, CH), jnp.int32),
        pltpu.VMEM((CH, 2 * F), jnp.float32),
        pltpu.VMEM((CH, 2 * F), jnp.float32),
        pltpu.VMEM((CH, F), jnp.float32),
        pltpu.VMEM_SHARED((NROWS, ROWW), jnp.float32),
        pltpu.SemaphoreType.DMA,
        pltpu.SemaphoreType.DMA,
        pltpu.SemaphoreType.DMA,
        pltpu.SemaphoreType.DMA,
        pltpu.SemaphoreType.DMA,
    ],
)
def _sc1(src_hbm, dst_hbm, ash_hbm, adxp_hbm, zeros_hbm, out_hbm,
         src_all, dst_all, g1a, g1b, ad_v, tbl,
         sga, sgb, ssa, ssb, sad):
    cid = lax.axis_index("c")
    sid = lax.axis_index("s")
    wid = cid * NS + sid
    r0 = pl.multiple_of(sid * RPT, 8)
    pltpu.sync_copy(zeros_hbm.at[pl.ds(r0, RPT)], tbl.at[pl.ds(r0, RPT)])
    pltpu.sync_copy(src_hbm.at[wid], src_all)
    pltpu.sync_copy(dst_hbm.at[wid], dst_all)
    plsc.subcore_barrier()

    def fire_g(i, g1, sem):
        pltpu.async_copy(ash_hbm.at[src_all.at[i]], g1, sem)

    def wait_g(i, g1, sem):
        pltpu.make_async_copy(ash_hbm.at[src_all.at[i]], g1, sem).wait()

    def fire_ad(i):
        pltpu.async_copy(adxp_hbm.at[dst_all.at[i]], ad_v, sad)

    def wait_ad(i):
        pltpu.make_async_copy(adxp_hbm.at[dst_all.at[i]], ad_v, sad).wait()

    def compute(g1):
        def edge(r, carry):
            for rr in range(2):
                for k in range(F // 16):
                    a = (g1[2 * r + rr, pl.ds(16 * k, 16)]
                         + ad_v[2 * r + rr, pl.ds(16 * k, 16)])
                    s = jnp.exp(jnp.maximum(a, 0.2 * a))
                    g1[2 * r + rr, pl.ds(16 * k, 16)] = s
                    g1[2 * r + rr, pl.ds(F + 16 * k, 16)] = (
                        s * g1[2 * r + rr, pl.ds(F + 16 * k, 16)])
            return carry

        lax.fori_loop(0, CH // 2, edge, 0)

    def fire_s(i, g1, sem):
        pltpu.async_copy(g1, tbl.at[dst_all.at[i]], sem, add=True)

    def wait_s(i, g1, sem):
        pltpu.make_async_copy(g1, tbl.at[dst_all.at[i]], sem).wait()

    fire_g(0, g1a, sga)
    fire_ad(0)

    def phase(i, g1x, semgx, semsx, g1y, semgy, semsy, first):
        wait_g(i, g1x, semgx)
        wait_ad(i)
        if not first:
            wait_s(i - 1, g1y, semsy)
        more = jnp.asarray(i + 1 < CHUNKS)

        @pl.when(more)
        def _():
            fire_g(i + 1, g1y, semgy)

        compute(g1x)

        @pl.when(more)
        def _():
            fire_ad(i + 1)

        fire_s(i, g1x, semsx)

    def body(j, carry):
        i0 = 2 * j
        phase(i0, g1a, sga, ssa, g1b, sgb, ssb, False)
        phase(i0 + 1, g1b, sgb, ssb, g1a, sga, ssa, False)
        return carry

    phase(0, g1a, sga, ssa, g1b, sgb, ssb, True)
    phase(1, g1b, sgb, ssb, g1a, sga, ssa, False)
    lax.fori_loop(1, CHUNKS // 2, body, 0)
    wait_s(CHUNKS - 1, g1b, ssb)
    plsc.subcore_barrier()
    pltpu.sync_copy(tbl.at[pl.ds(r0, RPT)], out_hbm.at[cid, pl.ds(r0, RPT)])


@functools.partial(
    pl.kernel,
    out_type=(jax.ShapeDtypeStruct((NC, NROWS, F), jnp.float32),
              jax.ShapeDtypeStruct((NC, NROWS, 16), jnp.float32)),
    mesh=_mesh,
    compiler_params=_sc_params,
    scratch_types=[
        pltpu.VMEM((CHUNKS, CH), jnp.int32),
        pltpu.VMEM((CHUNKS, CH), jnp.int32),
        pltpu.VMEM((CH, F), jnp.float32),
        pltpu.VMEM((CH, F), jnp.float32),
        pltpu.VMEM((CH, F), jnp.float32),
        pltpu.VMEM((CH, F), jnp.float32),
        pltpu.VMEM((CH, 16), jnp.float32),
        pltpu.VMEM_SHARED((NROWS, F), jnp.float32),
        pltpu.VMEM_SHARED((NROWS, 16), jnp.float32),
        pltpu.SemaphoreType.DMA,
        pltpu.SemaphoreType.DMA,
        pltpu.SemaphoreType.DMA,
        pltpu.SemaphoreType.DMA,
        pltpu.SemaphoreType.DMA,
        pltpu.SemaphoreType.DMA,
        pltpu.SemaphoreType.DMA,
        pltpu.SemaphoreType.DMA,
    ],
)
def _sc2(src_hbm, dst_hbm, x32_hbm, zeros_hbm, z16_hbm, out_hbm, deg_hbm,
         src_all, dst_all, g0, g1, g2, g3, ones_v, tbl, dtbl,
         sg0, sg1, sg2, sg3, ss0, ss1, ss2, ss3):
    cid = lax.axis_index("c")
    sid = lax.axis_index("s")
    wid = cid * NS + sid
    r0 = pl.multiple_of(sid * RPT, 8)
    pltpu.sync_copy(zeros_hbm.at[pl.ds(r0, RPT)], tbl.at[pl.ds(r0, RPT)])
    pltpu.sync_copy(z16_hbm.at[pl.ds(r0, RPT)], dtbl.at[pl.ds(r0, RPT)])
    pltpu.sync_copy(src_hbm.at[wid], src_all)
    pltpu.sync_copy(dst_hbm.at[wid], dst_all)

    one_hot = jnp.where(lax.iota(jnp.int32, 16) == 0, 1.0, 0.0)

    def preset(r, carry):
        ones_v[r, pl.ds(0, 16)] = one_hot
        return carry

    lax.fori_loop(0, CH, preset, 0)
    plsc.subcore_barrier()

    gbufs = (g0, g1, g2, g3)
    gsems = (sg0, sg1, sg2, sg3)
    ssems = (ss0, ss1, ss2, ss3)

    def fire_g(i, b, sem):
        pltpu.async_copy(x32_hbm.at[src_all.at[i]], b, sem)

    def wait_g(i, b, sem):
        pltpu.make_async_copy(x32_hbm.at[src_all.at[i]], b, sem).wait()

    def fire_s(i, b, sem):
        pltpu.async_copy(b, tbl.at[dst_all.at[i]], sem, add=True)
        pltpu.async_copy(ones_v, dtbl.at[dst_all.at[i]], sem, add=True)

    def wait_s(i, b, sem):
        pltpu.make_async_copy(b, tbl.at[dst_all.at[i]], sem).wait()
        pltpu.make_async_copy(ones_v, dtbl.at[dst_all.at[i]], sem).wait()

    for b in range(4):
        fire_g(b, gbufs[b], gsems[b])

    def body(j, carry):
        i0 = 4 * j
        for b in range(4):
            i = i0 + b
            wait_g(i, gbufs[b], gsems[b])
            fire_s(i, gbufs[b], ssems[b])
        for b in range(4):
            i = i0 + b

            @pl.when(i + 4 < CHUNKS)
            def _():
                wait_s(i, gbufs[b], ssems[b])
                fire_g(i + 4, gbufs[b], gsems[b])

        return carry

    lax.fori_loop(0, CHUNKS // 4, body, 0)
    for b in range(4):
        wait_s(CHUNKS - 4 + b, gbufs[b], ssems[b])
    plsc.subcore_barrier()
    pltpu.sync_copy(tbl.at[pl.ds(r0, RPT)], out_hbm.at[cid, pl.ds(r0, RPT)])
    pltpu.sync_copy(dtbl.at[pl.ds(r0, RPT)], deg_hbm.at[cid, pl.ds(r0, RPT)])


def _tca_body(x_ref, wg_ref, ws_ref, wd_ref,
              ash_ref, adx_ref, sxsl_ref, usl_ref):
    xb = x_ref[...]
    h = jnp.dot(xb, wg_ref[...], preferred_element_type=jnp.float32)
    asx = jnp.dot(xb, ws_ref[...], preferred_element_type=jnp.float32)
    adx = jnp.dot(xb, wd_ref[...], preferred_element_type=jnp.float32)
    e = asx + adx
    sx = jnp.exp(jnp.maximum(e, 0.2 * e))
    ash_ref[...] = jnp.concatenate([asx, h], axis=-1)
    adx_ref[...] = adx
    sxsl_ref[...] = sx
    usl_ref[...] = h * sx


def _tcb_body(p_ref, sxsl_ref, usl_ref, bg_ref, x32_ref):
    den = p_ref[0, :, 0:F] + p_ref[1, :, 0:F] + sxsl_ref[...]
    u = p_ref[0, :, F:2 * F] + p_ref[1, :, F:2 * F] + usl_ref[...]
    x32_ref[...] = jnp.maximum(u / den + bg_ref[...], 0.0)


def _tcc_body(aggp_ref, degp_ref, x32_ref, wlt_ref, wrt_ref, bs_ref, out_ref):
    deg = degp_ref[0, :, 0:1] + degp_ref[1, :, 0:1]
    inv = jnp.broadcast_to(1.0 / jnp.maximum(deg, 1.0), (BN, F))
    mean = (aggp_ref[0] + aggp_ref[1]) * inv
    out_ref[...] = (
        jnp.dot(mean, wlt_ref[...], preferred_element_type=jnp.float32)
        + jnp.dot(x32_ref[...], wrt_ref[...], preferred_element_type=jnp.float32)
        + bs_ref[...])


def kernel(x, edge_index, W1, b1, Wg, att_src, att_dst, bg, Wl, Wr, bs):
    eye8 = jnp.eye(HEADS, dtype=jnp.float32)
    ones8 = jnp.ones((1, 1, 1, HID), dtype=jnp.float32)
    m_s = (att_src[:, :, None, None] * eye8[:, None, :, None] * ones8
           ).reshape(F, F)
    m_d = (att_dst[:, :, None, None] * eye8[:, None, :, None] * ones8
           ).reshape(F, F)
    ws_x = Wg @ m_s
    wd_x = Wg @ m_d

    src = jnp.concatenate(
        [edge_index[0], jnp.zeros((E_PAD - E,), edge_index.dtype)])
    dst = jnp.concatenate(
        [edge_index[1], jnp.full((E_PAD - E,), N, edge_index.dtype)])
    src3 = src.astype(jnp.int32).reshape(NW, CHUNKS, CH)
    dst3 = dst.astype(jnp.int32).reshape(NW, CHUNKS, CH)
    z128 = jnp.zeros((NROWS, ROWW), jnp.float32)
    z64 = jnp.zeros((NROWS, F), jnp.float32)
    z16 = jnp.zeros((NROWS, 16), jnp.float32)

    ash, adx, sx_sl, u_sl = pl.pallas_call(
        _tca_body,
        grid=(GN,),
        in_specs=[
            pl.BlockSpec((BN, IN), lambda i: (i, 0)),
            pl.BlockSpec((IN, F), lambda i: (0, 0)),
            pl.BlockSpec((IN, F), lambda i: (0, 0)),
            pl.BlockSpec((IN, F), lambda i: (0, 0)),
        ],
        out_specs=[pl.BlockSpec((BN, 2 * F), lambda i: (i, 0))]
        + [pl.BlockSpec((BN, F), lambda i: (i, 0))] * 3,
        out_shape=[jax.ShapeDtypeStruct((N, 2 * F), jnp.float32)]
        + [jax.ShapeDtypeStruct((N, F), jnp.float32)] * 3,
    )(x, Wg, ws_x, wd_x)

    adx_p = jnp.concatenate([adx, jnp.zeros((1, F), jnp.float32)])

    part1 = _sc1(src3, dst3, ash, adx_p, z128)

    x32 = pl.pallas_call(
        _tcb_body,
        grid=(GN,),
        in_specs=[
            pl.BlockSpec((NC, BN, ROWW), lambda i: (0, i, 0)),
            pl.BlockSpec((BN, F), lambda i: (i, 0)),
            pl.BlockSpec((BN, F), lambda i: (i, 0)),
            pl.BlockSpec((1, F), lambda i: (0, 0)),
        ],
        out_specs=pl.BlockSpec((BN, F), lambda i: (i, 0)),
        out_shape=jax.ShapeDtypeStruct((N, F), jnp.float32),
    )(part1, sx_sl, u_sl, bg.reshape(1, F))

    part2, degp = _sc2(src3, dst3, x32, z64, z16)

    out = pl.pallas_call(
        _tcc_body,
        grid=(GN,),
        in_specs=[
            pl.BlockSpec((NC, BN, F), lambda i: (0, i, 0)),
            pl.BlockSpec((NC, BN, 16), lambda i: (0, i, 0)),
            pl.BlockSpec((BN, F), lambda i: (i, 0)),
            pl.BlockSpec((F, F), lambda i: (0, 0)),
            pl.BlockSpec((F, F), lambda i: (0, 0)),
            pl.BlockSpec((1, F), lambda i: (0, 0)),
        ],
        out_specs=pl.BlockSpec((BN, F), lambda i: (i, 0)),
        out_shape=jax.ShapeDtypeStruct((N, F), jnp.float32),
    )(part2, degp, x32, Wl.T, Wr.T, bs.reshape(1, F))

    return (x32, out)


# speedup vs baseline: 56.7978x; 1.5037x over previous
import functools

import jax
import jax.numpy as jnp
from jax import lax
from jax.experimental import pallas as pl
from jax.experimental.pallas import tpu as pltpu
from jax.experimental.pallas import tpu_sc as plsc

N = 10000
E = 320000
IN = 128
HID = 8
HEADS = 8
F = 64
ROWW = 80

NC = 2
NS = 16
NW = NC * NS
CH = 128
CHUNKS = 80
E_PAD = NW * CHUNKS * CH
RPT = 8 * (-(-(N + 1) // (NS * 8)))
NROWS = RPT * NS
BN = 400
GN = N // BN

_mesh = plsc.VectorSubcoreMesh(
    core_axis_name="c", subcore_axis_name="s", num_cores=NC, num_subcores=NS)
_sc_params = pltpu.CompilerParams(use_tc_tiling_on_sc=False)


@functools.partial(
    pl.kernel,
    out_type=jax.ShapeDtypeStruct((NC, NROWS, ROWW), jnp.float32),
    mesh=_mesh,
    compiler_params=_sc_params,
    scratch_types=[
        pltpu.VMEM((CHUNKS, CH), jnp.int32),
        pltpu.VMEM((CHUNKS, CH), jnp.int32),
        pltpu.VMEM((CH, 16), jnp.float32),
        pltpu.VMEM((CH, 16), jnp.float32),
        pltpu.VMEM((CH, 16), jnp.float32),
        pltpu.VMEM((CH, 16), jnp.float32),
        pltpu.VMEM((CH, F), jnp.float32),
        pltpu.VMEM((CH, F), jnp.float32),
        pltpu.VMEM((CH, ROWW), jnp.float32),
        pltpu.VMEM((CH, ROWW), jnp.float32),
        pltpu.VMEM_SHARED((NROWS, ROWW), jnp.float32),
        pltpu.SemaphoreType.DMA,
        pltpu.SemaphoreType.DMA,
        pltpu.SemaphoreType.DMA,
        pltpu.SemaphoreType.DMA,
    ],
)
def _sc1(src_hbm, dst_hbm, as16_hbm, ad16_hbm, h_hbm, zeros_hbm, out_hbm,
         src_all, dst_all, saa, sab, ada, adb, ha, hb, rowa, rowb, tbl,
         sga, sgb, ssa, ssb):
    cid = lax.axis_index("c")
    sid = lax.axis_index("s")
    wid = cid * NS + sid
    r0 = pl.multiple_of(sid * RPT, 8)
    pltpu.sync_copy(zeros_hbm.at[pl.ds(r0, RPT)], tbl.at[pl.ds(r0, RPT)])
    pltpu.sync_copy(src_hbm.at[wid], src_all)
    pltpu.sync_copy(dst_hbm.at[wid], dst_all)
    plsc.subcore_barrier()

    def fire_g(i, sa, ad, h, sem):
        pltpu.async_copy(as16_hbm.at[src_all.at[i]], sa, sem)
        pltpu.async_copy(ad16_hbm.at[dst_all.at[i]], ad, sem)
        pltpu.async_copy(h_hbm.at[src_all.at[i]], h, sem)

    def wait_g(i, sa, ad, h, sem):
        pltpu.make_async_copy(as16_hbm.at[src_all.at[i]], sa, sem).wait()
        pltpu.make_async_copy(ad16_hbm.at[dst_all.at[i]], ad, sem).wait()
        pltpu.make_async_copy(h_hbm.at[src_all.at[i]], h, sem).wait()

    lane = lax.iota(jnp.int32, 16)
    perm_idx = [jnp.where(lane < 8, 2 * k, 2 * k + 1) for k in range(4)]
    _dnums = lax.GatherDimensionNumbers(
        offset_dims=(), collapsed_slice_dims=(0,), start_index_map=(0,))

    def dyn_gather(v, idxv):
        return lax.gather(
            v, idxv[:, None], _dnums, slice_sizes=(1,),
            mode=lax.GatherScatterMode.PROMISE_IN_BOUNDS)

    def compute(sa, ad, h, row):
        def edge(r, carry):
            a = sa[r, pl.ds(0, 16)] + ad[r, pl.ds(0, 16)]
            s = jnp.exp(jnp.maximum(a, 0.2 * a))
            row[r, pl.ds(F, 16)] = s
            for k in range(4):
                m = dyn_gather(s, perm_idx[k])
                row[r, pl.ds(16 * k, 16)] = m * h[r, pl.ds(16 * k, 16)]
            return carry

        lax.fori_loop(0, CH, edge, 0)

    def fire_s(i, row, sem):
        pltpu.async_copy(row, tbl.at[dst_all.at[i]], sem, add=True)

    def wait_s(i, row, sem):
        pltpu.make_async_copy(row, tbl.at[dst_all.at[i]], sem).wait()

    fire_g(0, saa, ada, ha, sga)

    def phase(i, sax, adx, hx, rowx, semgx, semsx,
              say, ady, hy, rowy, semgy, semsy, first):
        wait_g(i, sax, adx, hx, semgx)
        if not first:
            wait_s(i - 1, rowy, semsy)
        more = jnp.asarray(i + 1 < CHUNKS)

        @pl.when(more)
        def _():
            fire_g(i + 1, say, ady, hy, semgy)

        compute(sax, adx, hx, rowx)
        fire_s(i, rowx, semsx)

    def body(j, carry):
        i0 = 2 * j
        phase(i0, saa, ada, ha, rowa, sga, ssa,
              sab, adb, hb, rowb, sgb, ssb, False)
        phase(i0 + 1, sab, adb, hb, rowb, sgb, ssb,
              saa, ada, ha, rowa, sga, ssa, False)
        return carry

    phase(0, saa, ada, ha, rowa, sga, ssa,
          sab, adb, hb, rowb, sgb, ssb, True)
    phase(1, sab, adb, hb, rowb, sgb, ssb,
          saa, ada, ha, rowa, sga, ssa, False)
    lax.fori_loop(1, CHUNKS // 2, body, 0)
    wait_s(CHUNKS - 1, rowb, ssb)
    plsc.subcore_barrier()
    pltpu.sync_copy(tbl.at[pl.ds(r0, RPT)], out_hbm.at[cid, pl.ds(r0, RPT)])


@functools.partial(
    pl.kernel,
    out_type=(jax.ShapeDtypeStruct((NC, NROWS, F), jnp.float32),
              jax.ShapeDtypeStruct((NC, NROWS, 16), jnp.float32)),
    mesh=_mesh,
    compiler_params=_sc_params,
    scratch_types=[
        pltpu.VMEM((CHUNKS, CH), jnp.int32),
        pltpu.VMEM((CHUNKS, CH), jnp.int32),
        pltpu.VMEM((CH, F), jnp.float32),
        pltpu.VMEM((CH, F), jnp.float32),
        pltpu.VMEM((CH, F), jnp.float32),
        pltpu.VMEM((CH, F), jnp.float32),
        pltpu.VMEM((CH, 16), jnp.float32),
        pltpu.VMEM_SHARED((NROWS, F), jnp.float32),
        pltpu.VMEM_SHARED((NROWS, 16), jnp.float32),
        pltpu.SemaphoreType.DMA,
        pltpu.SemaphoreType.DMA,
        pltpu.SemaphoreType.DMA,
        pltpu.SemaphoreType.DMA,
        pltpu.SemaphoreType.DMA,
        pltpu.SemaphoreType.DMA,
        pltpu.SemaphoreType.DMA,
        pltpu.SemaphoreType.DMA,
    ],
)
def _sc2(src_hbm, dst_hbm, x32_hbm, zeros_hbm, z16_hbm, out_hbm, deg_hbm,
         src_all, dst_all, g0, g1, g2, g3, ones_v, tbl, dtbl,
         sg0, sg1, sg2, sg3, ss0, ss1, ss2, ss3):
    cid = lax.axis_index("c")
    sid = lax.axis_index("s")
    wid = cid * NS + sid
    r0 = pl.multiple_of(sid * RPT, 8)
    pltpu.sync_copy(zeros_hbm.at[pl.ds(r0, RPT)], tbl.at[pl.ds(r0, RPT)])
    pltpu.sync_copy(z16_hbm.at[pl.ds(r0, RPT)], dtbl.at[pl.ds(r0, RPT)])
    pltpu.sync_copy(src_hbm.at[wid], src_all)
    pltpu.sync_copy(dst_hbm.at[wid], dst_all)

    one_hot = jnp.where(lax.iota(jnp.int32, 16) == 0, 1.0, 0.0)

    def preset(r, carry):
        ones_v[r, pl.ds(0, 16)] = one_hot
        return carry

    lax.fori_loop(0, CH, preset, 0)
    plsc.subcore_barrier()

    gbufs = (g0, g1, g2, g3)
    gsems = (sg0, sg1, sg2, sg3)
    ssems = (ss0, ss1, ss2, ss3)

    def fire_g(i, b, sem):
        pltpu.async_copy(x32_hbm.at[src_all.at[i]], b, sem)

    def wait_g(i, b, sem):
        pltpu.make_async_copy(x32_hbm.at[src_all.at[i]], b, sem).wait()

    def fire_s(i, b, sem):
        pltpu.async_copy(b, tbl.at[dst_all.at[i]], sem, add=True)
        pltpu.async_copy(ones_v, dtbl.at[dst_all.at[i]], sem, add=True)

    def wait_s(i, b, sem):
        pltpu.make_async_copy(b, tbl.at[dst_all.at[i]], sem).wait()
        pltpu.make_async_copy(ones_v, dtbl.at[dst_all.at[i]], sem).wait()

    for b in range(4):
        fire_g(b, gbufs[b], gsems[b])

    def body(j, carry):
        i0 = 4 * j
        for b in range(4):
            i = i0 + b
            wait_g(i, gbufs[b], gsems[b])
            fire_s(i, gbufs[b], ssems[b])
        for b in range(4):
            i = i0 + b

            @pl.when(jnp.asarray(i + 4 < CHUNKS))
            def _():
                wait_s(i, gbufs[b], ssems[b])
                fire_g(i + 4, gbufs[b], gsems[b])

        return carry

    lax.fori_loop(0, CHUNKS // 4, body, 0)
    for b in range(4):
        wait_s(CHUNKS - 4 + b, gbufs[b], ssems[b])
    plsc.subcore_barrier()
    pltpu.sync_copy(tbl.at[pl.ds(r0, RPT)], out_hbm.at[cid, pl.ds(r0, RPT)])
    pltpu.sync_copy(dtbl.at[pl.ds(r0, RPT)], deg_hbm.at[cid, pl.ds(r0, RPT)])


def _tca_body(x_ref, wg_ref, as_ref, ad_ref, k_ref,
              h_ref, as16_ref, ad16_ref, ssl16_ref, usl_ref):
    xb = x_ref[...]
    h = jnp.dot(xb, wg_ref[...], preferred_element_type=jnp.float32)
    a_s = jnp.dot(h, as_ref[...], preferred_element_type=jnp.float32)
    a_d = jnp.dot(h, ad_ref[...], preferred_element_type=jnp.float32)
    e = a_s + a_d
    s8 = jnp.exp(jnp.maximum(e, 0.2 * e))
    z8 = jnp.zeros((BN, HEADS), jnp.float32)
    h_ref[...] = h
    as16_ref[...] = jnp.concatenate([a_s, z8], axis=-1)
    ad16_ref[...] = jnp.concatenate([a_d, z8], axis=-1)
    ssl16_ref[...] = jnp.concatenate([s8, z8], axis=-1)
    sx64 = jnp.dot(s8, k_ref[...], preferred_element_type=jnp.float32)
    usl_ref[...] = h * sx64


def _tcb_body(p_ref, ssl16_ref, usl_ref, k_ref, bg_ref, x32_ref):
    den8 = (p_ref[0, :, F:F + HEADS] + p_ref[1, :, F:F + HEADS]
            + ssl16_ref[:, 0:HEADS])
    den = jnp.dot(den8, k_ref[...], preferred_element_type=jnp.float32)
    u = p_ref[0, :, 0:F] + p_ref[1, :, 0:F] + usl_ref[...]
    x32_ref[...] = jnp.maximum(u / den + bg_ref[...], 0.0)


def _tcc_body(aggp_ref, degp_ref, x32_ref, wlt_ref, wrt_ref, bs_ref, out_ref):
    deg = degp_ref[0, :, 0:1] + degp_ref[1, :, 0:1]
    inv = jnp.broadcast_to(1.0 / jnp.maximum(deg, 1.0), (BN, F))
    mean = (aggp_ref[0] + aggp_ref[1]) * inv
    out_ref[...] = (
        jnp.dot(mean, wlt_ref[...], preferred_element_type=jnp.float32)
        + jnp.dot(x32_ref[...], wrt_ref[...], preferred_element_type=jnp.float32)
        + bs_ref[...])


def kernel(x, edge_index, W1, b1, Wg, att_src, att_dst, bg, Wl, Wr, bs):
    idx = jnp.arange(F)
    a_s_m = jnp.zeros((F, HEADS), jnp.float32).at[
        idx, idx // HID].set(att_src.reshape(-1))
    a_d_m = jnp.zeros((F, HEADS), jnp.float32).at[
        idx, idx // HID].set(att_dst.reshape(-1))
    k_exp = jnp.repeat(jnp.eye(HEADS, dtype=jnp.float32), HID, axis=1)

    src = jnp.concatenate(
        [edge_index[0], jnp.zeros((E_PAD - E,), edge_index.dtype)])
    dst = jnp.concatenate(
        [edge_index[1], jnp.full((E_PAD - E,), N, edge_index.dtype)])
    src3 = src.astype(jnp.int32).reshape(NW, CHUNKS, CH)
    dst3 = dst.astype(jnp.int32).reshape(NW, CHUNKS, CH)
    z80 = jnp.zeros((NROWS, ROWW), jnp.float32)
    z64 = jnp.zeros((NROWS, F), jnp.float32)
    z16 = jnp.zeros((NROWS, 16), jnp.float32)

    h, as16, ad16, ssl16, u_sl = pl.pallas_call(
        _tca_body,
        grid=(GN,),
        in_specs=[
            pl.BlockSpec((BN, IN), lambda i: (i, 0)),
            pl.BlockSpec((IN, F), lambda i: (0, 0)),
            pl.BlockSpec((F, HEADS), lambda i: (0, 0)),
            pl.BlockSpec((F, HEADS), lambda i: (0, 0)),
            pl.BlockSpec((HEADS, F), lambda i: (0, 0)),
        ],
        out_specs=[
            pl.BlockSpec((BN, F), lambda i: (i, 0)),
            pl.BlockSpec((BN, 16), lambda i: (i, 0)),
            pl.BlockSpec((BN, 16), lambda i: (i, 0)),
            pl.BlockSpec((BN, 16), lambda i: (i, 0)),
            pl.BlockSpec((BN, F), lambda i: (i, 0)),
        ],
        out_shape=[
            jax.ShapeDtypeStruct((N, F), jnp.float32),
            jax.ShapeDtypeStruct((N, 16), jnp.float32),
            jax.ShapeDtypeStruct((N, 16), jnp.float32),
            jax.ShapeDtypeStruct((N, 16), jnp.float32),
            jax.ShapeDtypeStruct((N, F), jnp.float32),
        ],
    )(x, Wg, a_s_m, a_d_m, k_exp)

    ad16_p = jnp.concatenate([ad16, jnp.zeros((1, 16), jnp.float32)])

    part1 = _sc1(src3, dst3, as16, ad16_p, h, z80)

    x32 = pl.pallas_call(
        _tcb_body,
        grid=(GN,),
        in_specs=[
            pl.BlockSpec((NC, BN, ROWW), lambda i: (0, i, 0)),
            pl.BlockSpec((BN, 16), lambda i: (i, 0)),
            pl.BlockSpec((BN, F), lambda i: (i, 0)),
            pl.BlockSpec((HEADS, F), lambda i: (0, 0)),
            pl.BlockSpec((1, F), lambda i: (0, 0)),
        ],
        out_specs=pl.BlockSpec((BN, F), lambda i: (i, 0)),
        out_shape=jax.ShapeDtypeStruct((N, F), jnp.float32),
    )(part1, ssl16, u_sl, k_exp, bg.reshape(1, F))

    part2, degp = _sc2(src3, dst3, x32, z64, z16)

    out = pl.pallas_call(
        _tcc_body,
        grid=(GN,),
        in_specs=[
            pl.BlockSpec((NC, BN, F), lambda i: (0, i, 0)),
            pl.BlockSpec((NC, BN, 16), lambda i: (0, i, 0)),
            pl.BlockSpec((BN, F), lambda i: (i, 0)),
            pl.BlockSpec((F, F), lambda i: (0, 0)),
            pl.BlockSpec((F, F), lambda i: (0, 0)),
            pl.BlockSpec((1, F), lambda i: (0, 0)),
        ],
        out_specs=pl.BlockSpec((BN, F), lambda i: (i, 0)),
        out_shape=jax.ShapeDtypeStruct((N, F), jnp.float32),
    )(part2, degp, x32, Wl.T, Wr.T, bs.reshape(1, F))

    return (x32, out)


# speedup vs baseline: 57.9601x; 1.0205x over previous
import functools

import jax
import jax.numpy as jnp
from jax import lax
from jax.experimental import pallas as pl
from jax.experimental.pallas import tpu as pltpu
from jax.experimental.pallas import tpu_sc as plsc

N = 10000
E = 320000
IN = 128
HID = 8
HEADS = 8
F = 64
ROWW = 80

NC = 2
NS = 16
NW = NC * NS
CH = 128
CHUNKS = 80
E_PAD = NW * CHUNKS * CH
RPT = 8 * (-(-(N + 1) // (NS * 8)))
NROWS = RPT * NS
BN = 400
GN = N // BN

_mesh = plsc.VectorSubcoreMesh(
    core_axis_name="c", subcore_axis_name="s", num_cores=NC, num_subcores=NS)
_sc_params = pltpu.CompilerParams(use_tc_tiling_on_sc=False)


@functools.partial(
    pl.kernel,
    out_type=jax.ShapeDtypeStruct((NC, NROWS, ROWW), jnp.float32),
    mesh=_mesh,
    compiler_params=_sc_params,
    scratch_types=[
        pltpu.VMEM((CHUNKS, CH), jnp.int32),
        pltpu.VMEM((CHUNKS, CH), jnp.int32),
        pltpu.VMEM((CH, 16), jnp.float32),
        pltpu.VMEM((CH, 16), jnp.float32),
        pltpu.VMEM((CH, 16), jnp.float32),
        pltpu.VMEM((CH, 16), jnp.float32),
        pltpu.VMEM((CH, F), jnp.float32),
        pltpu.VMEM((CH, F), jnp.float32),
        pltpu.VMEM((CH, ROWW), jnp.float32),
        pltpu.VMEM((CH, ROWW), jnp.float32),
        pltpu.VMEM_SHARED((NROWS, ROWW), jnp.float32),
        pltpu.SemaphoreType.DMA,
        pltpu.SemaphoreType.DMA,
        pltpu.SemaphoreType.DMA,
        pltpu.SemaphoreType.DMA,
    ],
)
def _sc1(src_hbm, dst_hbm, as16_hbm, ad16_hbm, h_hbm, zeros_hbm, out_hbm,
         src_all, dst_all, saa, sab, ada, adb, ha, hb, rowa, rowb, tbl,
         sga, sgb, ssa, ssb):
    cid = lax.axis_index("c")
    sid = lax.axis_index("s")
    wid = cid * NS + sid
    r0 = pl.multiple_of(sid * RPT, 8)
    pltpu.sync_copy(zeros_hbm.at[pl.ds(r0, RPT)], tbl.at[pl.ds(r0, RPT)])
    pltpu.sync_copy(src_hbm.at[wid], src_all)
    pltpu.sync_copy(dst_hbm.at[wid], dst_all)
    plsc.subcore_barrier()

    def fire_g(i, sa, ad, h, sem):
        pltpu.async_copy(as16_hbm.at[src_all.at[i]], sa, sem)
        pltpu.async_copy(ad16_hbm.at[dst_all.at[i]], ad, sem)
        pltpu.async_copy(h_hbm.at[src_all.at[i]], h, sem)

    def wait_g(i, sa, ad, h, sem):
        pltpu.make_async_copy(as16_hbm.at[src_all.at[i]], sa, sem).wait()
        pltpu.make_async_copy(ad16_hbm.at[dst_all.at[i]], ad, sem).wait()
        pltpu.make_async_copy(h_hbm.at[src_all.at[i]], h, sem).wait()

    lane = lax.iota(jnp.int32, 16)
    perm_idx = [jnp.where(lane < 8, 2 * k, 2 * k + 1) for k in range(4)]
    _dnums = lax.GatherDimensionNumbers(
        offset_dims=(), collapsed_slice_dims=(0,), start_index_map=(0,))

    def dyn_gather(v, idxv):
        return lax.gather(
            v, idxv[:, None], _dnums, slice_sizes=(1,),
            mode=lax.GatherScatterMode.PROMISE_IN_BOUNDS)

    def compute(sa, ad, h, row):
        def edge(r, carry):
            a = sa[r, pl.ds(0, 16)] + ad[r, pl.ds(0, 16)]
            s = jnp.exp(jnp.maximum(a, 0.2 * a))
            row[r, pl.ds(F, 16)] = s
            for k in range(4):
                m = dyn_gather(s, perm_idx[k])
                row[r, pl.ds(16 * k, 16)] = m * h[r, pl.ds(16 * k, 16)]
            return carry

        lax.fori_loop(0, CH, edge, 0)

    def fire_s(i, row, sem):
        pltpu.async_copy(row, tbl.at[dst_all.at[i]], sem, add=True)

    def wait_s(i, row, sem):
        pltpu.make_async_copy(row, tbl.at[dst_all.at[i]], sem).wait()

    fire_g(0, saa, ada, ha, sga)

    def phase(i, sax, adx, hx, rowx, semgx, semsx,
              say, ady, hy, rowy, semgy, semsy, first):
        wait_g(i, sax, adx, hx, semgx)
        if not first:
            wait_s(i - 1, rowy, semsy)
        more = jnp.asarray(i + 1 < CHUNKS)

        @pl.when(more)
        def _():
            fire_g(i + 1, say, ady, hy, semgy)

        compute(sax, adx, hx, rowx)
        fire_s(i, rowx, semsx)

    def body(j, carry):
        i0 = 2 * j
        phase(i0, saa, ada, ha, rowa, sga, ssa,
              sab, adb, hb, rowb, sgb, ssb, False)
        phase(i0 + 1, sab, adb, hb, rowb, sgb, ssb,
              saa, ada, ha, rowa, sga, ssa, False)
        return carry

    phase(0, saa, ada, ha, rowa, sga, ssa,
          sab, adb, hb, rowb, sgb, ssb, True)
    phase(1, sab, adb, hb, rowb, sgb, ssb,
          saa, ada, ha, rowa, sga, ssa, False)
    lax.fori_loop(1, CHUNKS // 2, body, 0)
    wait_s(CHUNKS - 1, rowb, ssb)
    plsc.subcore_barrier()
    pltpu.sync_copy(tbl.at[pl.ds(r0, RPT)], out_hbm.at[cid, pl.ds(r0, RPT)])


@functools.partial(
    pl.kernel,
    out_type=jax.ShapeDtypeStruct((NC, NROWS, F), jnp.float32),
    mesh=_mesh,
    compiler_params=_sc_params,
    scratch_types=[
        pltpu.VMEM((CHUNKS, CH), jnp.int32),
        pltpu.VMEM((CHUNKS, CH), jnp.int32),
        pltpu.VMEM((CH, F), jnp.float32),
        pltpu.VMEM((CH, F), jnp.float32),
        pltpu.VMEM((CH, F), jnp.float32),
        pltpu.VMEM((CH, F), jnp.float32),
        pltpu.VMEM_SHARED((NROWS, F), jnp.float32),
        pltpu.SemaphoreType.DMA,
        pltpu.SemaphoreType.DMA,
        pltpu.SemaphoreType.DMA,
        pltpu.SemaphoreType.DMA,
        pltpu.SemaphoreType.DMA,
        pltpu.SemaphoreType.DMA,
        pltpu.SemaphoreType.DMA,
        pltpu.SemaphoreType.DMA,
    ],
)
def _sc2(src_hbm, dst_hbm, x32_hbm, zeros_hbm, out_hbm,
         src_all, dst_all, g0, g1, g2, g3, tbl,
         sg0, sg1, sg2, sg3, ss0, ss1, ss2, ss3):
    cid = lax.axis_index("c")
    sid = lax.axis_index("s")
    wid = cid * NS + sid
    r0 = pl.multiple_of(sid * RPT, 8)
    pltpu.sync_copy(zeros_hbm.at[pl.ds(r0, RPT)], tbl.at[pl.ds(r0, RPT)])
    pltpu.sync_copy(src_hbm.at[wid], src_all)
    pltpu.sync_copy(dst_hbm.at[wid], dst_all)
    plsc.subcore_barrier()

    gbufs = (g0, g1, g2, g3)
    gsems = (sg0, sg1, sg2, sg3)
    ssems = (ss0, ss1, ss2, ss3)

    def fire_g(i, b, sem):
        pltpu.async_copy(x32_hbm.at[src_all.at[i]], b, sem)

    def wait_g(i, b, sem):
        pltpu.make_async_copy(x32_hbm.at[src_all.at[i]], b, sem).wait()

    def fire_s(i, b, sem):
        pltpu.async_copy(b, tbl.at[dst_all.at[i]], sem, add=True)

    def wait_s(i, b, sem):
        pltpu.make_async_copy(b, tbl.at[dst_all.at[i]], sem).wait()

    for b in range(4):
        fire_g(b, gbufs[b], gsems[b])

    def body(j, carry):
        i0 = 4 * j
        for b in range(4):
            i = i0 + b
            wait_g(i, gbufs[b], gsems[b])
            fire_s(i, gbufs[b], ssems[b])
        for b in range(4):
            i = i0 + b

            @pl.when(jnp.asarray(i + 4 < CHUNKS))
            def _():
                wait_s(i, gbufs[b], ssems[b])
                fire_g(i + 4, gbufs[b], gsems[b])

        return carry

    lax.fori_loop(0, CHUNKS // 4, body, 0)
    for b in range(4):
        wait_s(CHUNKS - 4 + b, gbufs[b], ssems[b])
    plsc.subcore_barrier()
    pltpu.sync_copy(tbl.at[pl.ds(r0, RPT)], out_hbm.at[cid, pl.ds(r0, RPT)])


def _tca_body(x_ref, wg_ref, as_ref, ad_ref, k_ref,
              h_ref, as16_ref, ad16_ref, ssl16_ref, usl_ref):
    xb = x_ref[...]
    h = jnp.dot(xb, wg_ref[...], preferred_element_type=jnp.float32)
    a_s = jnp.dot(h, as_ref[...], preferred_element_type=jnp.float32)
    a_d = jnp.dot(h, ad_ref[...], preferred_element_type=jnp.float32)
    e = a_s + a_d
    s8 = jnp.exp(jnp.maximum(e, 0.2 * e))
    z8 = jnp.zeros((BN, HEADS), jnp.float32)
    h_ref[...] = h
    as16_ref[...] = jnp.concatenate([a_s, z8], axis=-1)
    ad16_ref[...] = jnp.concatenate([a_d, z8], axis=-1)
    ssl16_ref[...] = jnp.concatenate([s8, z8], axis=-1)
    sx64 = jnp.dot(s8, k_ref[...], preferred_element_type=jnp.float32)
    usl_ref[...] = h * sx64


def _tcb_body(p_ref, ssl16_ref, usl_ref, k_ref, bg_ref, x32_ref, inv_ref):
    den8 = (p_ref[0, :, F:F + HEADS] + p_ref[1, :, F:F + HEADS]
            + ssl16_ref[:, 0:HEADS])
    den = jnp.dot(den8, k_ref[...], preferred_element_type=jnp.float32)
    u = p_ref[0, :, 0:F] + p_ref[1, :, 0:F] + usl_ref[...]
    x32_ref[...] = jnp.maximum(u / den + bg_ref[...], 0.0)
    deg = p_ref[0, :, 72:73] + p_ref[1, :, 72:73]
    inv_ref[...] = jnp.broadcast_to(1.0 / jnp.maximum(deg, 1.0), (BN, F))


def _tcc_body(aggp_ref, inv_ref, x32_ref, wlt_ref, wrt_ref, bs_ref, out_ref):
    mean = (aggp_ref[0] + aggp_ref[1]) * inv_ref[...]
    out_ref[...] = (
        jnp.dot(mean, wlt_ref[...], preferred_element_type=jnp.float32)
        + jnp.dot(x32_ref[...], wrt_ref[...], preferred_element_type=jnp.float32)
        + bs_ref[...])


def kernel(x, edge_index, W1, b1, Wg, att_src, att_dst, bg, Wl, Wr, bs):
    idx = jnp.arange(F)
    a_s_m = jnp.zeros((F, HEADS), jnp.float32).at[
        idx, idx // HID].set(att_src.reshape(-1))
    a_d_m = jnp.zeros((F, HEADS), jnp.float32).at[
        idx, idx // HID].set(att_dst.reshape(-1))
    k_exp = jnp.repeat(jnp.eye(HEADS, dtype=jnp.float32), HID, axis=1)

    src = jnp.concatenate(
        [edge_index[0], jnp.zeros((E_PAD - E,), edge_index.dtype)])
    dst = jnp.concatenate(
        [edge_index[1], jnp.full((E_PAD - E,), N, edge_index.dtype)])
    src3 = src.astype(jnp.int32).reshape(NW, CHUNKS, CH)
    dst3 = dst.astype(jnp.int32).reshape(NW, CHUNKS, CH)
    z80 = jnp.zeros((NROWS, ROWW), jnp.float32)
    z64 = jnp.zeros((NROWS, F), jnp.float32)

    h, as16, ad16, ssl16, u_sl = pl.pallas_call(
        _tca_body,
        grid=(GN,),
        in_specs=[
            pl.BlockSpec((BN, IN), lambda i: (i, 0)),
            pl.BlockSpec((IN, F), lambda i: (0, 0)),
            pl.BlockSpec((F, HEADS), lambda i: (0, 0)),
            pl.BlockSpec((F, HEADS), lambda i: (0, 0)),
            pl.BlockSpec((HEADS, F), lambda i: (0, 0)),
        ],
        out_specs=[
            pl.BlockSpec((BN, F), lambda i: (i, 0)),
            pl.BlockSpec((BN, 16), lambda i: (i, 0)),
            pl.BlockSpec((BN, 16), lambda i: (i, 0)),
            pl.BlockSpec((BN, 16), lambda i: (i, 0)),
            pl.BlockSpec((BN, F), lambda i: (i, 0)),
        ],
        out_shape=[
            jax.ShapeDtypeStruct((N, F), jnp.float32),
            jax.ShapeDtypeStruct((N, 16), jnp.float32),
            jax.ShapeDtypeStruct((N, 16), jnp.float32),
            jax.ShapeDtypeStruct((N, 16), jnp.float32),
            jax.ShapeDtypeStruct((N, F), jnp.float32),
        ],
    )(x, Wg, a_s_m, a_d_m, k_exp)

    ad16_p = jnp.concatenate([ad16, jnp.zeros((1, 16), jnp.float32)])

    part1 = _sc1(src3, dst3, as16, ad16_p, h, z80)

    x32, invd = pl.pallas_call(
        _tcb_body,
        grid=(GN,),
        in_specs=[
            pl.BlockSpec((NC, BN, ROWW), lambda i: (0, i, 0)),
            pl.BlockSpec((BN, 16), lambda i: (i, 0)),
            pl.BlockSpec((BN, F), lambda i: (i, 0)),
            pl.BlockSpec((HEADS, F), lambda i: (0, 0)),
            pl.BlockSpec((1, F), lambda i: (0, 0)),
        ],
        out_specs=[pl.BlockSpec((BN, F), lambda i: (i, 0))] * 2,
        out_shape=[jax.ShapeDtypeStruct((N, F), jnp.float32)] * 2,
    )(part1, ssl16, u_sl, k_exp, bg.reshape(1, F))

    part2 = _sc2(src3, dst3, x32, z64)

    out = pl.pallas_call(
        _tcc_body,
        grid=(GN,),
        in_specs=[
            pl.BlockSpec((NC, BN, F), lambda i: (0, i, 0)),
            pl.BlockSpec((BN, F), lambda i: (i, 0)),
            pl.BlockSpec((BN, F), lambda i: (i, 0)),
            pl.BlockSpec((F, F), lambda i: (0, 0)),
            pl.BlockSpec((F, F), lambda i: (0, 0)),
            pl.BlockSpec((1, F), lambda i: (0, 0)),
        ],
        out_specs=pl.BlockSpec((BN, F), lambda i: (i, 0)),
        out_shape=jax.ShapeDtypeStruct((N, F), jnp.float32),
    )(part2, invd, x32, Wl.T, Wr.T, bs.reshape(1, F))

    return (x32, out)


# speedup vs baseline: 75.2600x; 1.2985x over previous
import functools

import jax
import jax.numpy as jnp
from jax import lax
from jax.experimental import pallas as pl
from jax.experimental.pallas import tpu as pltpu
from jax.experimental.pallas import tpu_sc as plsc

N = 10000
E = 320000
IN = 128
HID = 8
HEADS = 8
F = 64
ROWW = 80

NC = 2
NS = 16
NW = NC * NS
CH = 128
CHUNKS = 80
E_PAD = NW * CHUNKS * CH
RPT = 8 * (-(-(N + 1) // (NS * 8)))
NROWS = RPT * NS
BN = 400
GN = N // BN

_mesh = plsc.VectorSubcoreMesh(
    core_axis_name="c", subcore_axis_name="s", num_cores=NC, num_subcores=NS)
_sc_params = pltpu.CompilerParams(use_tc_tiling_on_sc=False)


@functools.partial(
    pl.kernel,
    out_type=jax.ShapeDtypeStruct((NC, NROWS, ROWW), jnp.float32),
    mesh=_mesh,
    compiler_params=_sc_params,
    scratch_types=[
        pltpu.VMEM((CHUNKS, CH), jnp.int32),
        pltpu.VMEM((CHUNKS, CH), jnp.int32),
        pltpu.VMEM((CH, 16), jnp.float32),
        pltpu.VMEM((CH, 16), jnp.float32),
        pltpu.VMEM((CH, 16), jnp.float32),
        pltpu.VMEM((CH, 16), jnp.float32),
        pltpu.VMEM((CH, F), jnp.float32),
        pltpu.VMEM((CH, F), jnp.float32),
        pltpu.VMEM((CH, ROWW), jnp.float32),
        pltpu.VMEM((CH, ROWW), jnp.float32),
        pltpu.VMEM_SHARED((NROWS, ROWW), jnp.float32),
        pltpu.SemaphoreType.DMA,
        pltpu.SemaphoreType.DMA,
        pltpu.SemaphoreType.DMA,
        pltpu.SemaphoreType.DMA,
    ],
)
def _sc1(src_hbm, dst_hbm, as16_hbm, ad16_hbm, h_hbm, zeros_hbm, out_hbm,
         src_all, dst_all, saa, sab, ada, adb, ha, hb, rowa, rowb, tbl,
         sga, sgb, ssa, ssb):
    cid = lax.axis_index("c")
    sid = lax.axis_index("s")
    wid = cid * NS + sid
    r0 = pl.multiple_of(sid * RPT, 8)
    pltpu.sync_copy(zeros_hbm.at[pl.ds(r0, RPT)], tbl.at[pl.ds(r0, RPT)])
    pltpu.sync_copy(src_hbm.at[wid], src_all)
    pltpu.sync_copy(dst_hbm.at[wid], dst_all)
    plsc.subcore_barrier()

    def fire_g(i, sa, ad, h, sem):
        pltpu.async_copy(as16_hbm.at[src_all.at[i]], sa, sem)
        pltpu.async_copy(ad16_hbm.at[dst_all.at[i]], ad, sem)
        pltpu.async_copy(h_hbm.at[src_all.at[i]], h, sem)

    def wait_g(i, sa, ad, h, sem):
        pltpu.make_async_copy(as16_hbm.at[src_all.at[i]], sa, sem).wait()
        pltpu.make_async_copy(ad16_hbm.at[dst_all.at[i]], ad, sem).wait()
        pltpu.make_async_copy(h_hbm.at[src_all.at[i]], h, sem).wait()

    lane = lax.iota(jnp.int32, 16)
    perm_idx = [jnp.where(lane < 8, 2 * k, 2 * k + 1) for k in range(4)]
    _dnums = lax.GatherDimensionNumbers(
        offset_dims=(), collapsed_slice_dims=(0,), start_index_map=(0,))

    def dyn_gather(v, idxv):
        return lax.gather(
            v, idxv[:, None], _dnums, slice_sizes=(1,),
            mode=lax.GatherScatterMode.PROMISE_IN_BOUNDS)

    def compute(sa, ad, h, row):
        def edge(r, carry):
            a = sa[r, pl.ds(0, 16)] + ad[r, pl.ds(0, 16)]
            s = jnp.exp(jnp.maximum(a, 0.2 * a))
            row[r, pl.ds(F, 16)] = s
            for k in range(4):
                m = dyn_gather(s, perm_idx[k])
                row[r, pl.ds(16 * k, 16)] = m * h[r, pl.ds(16 * k, 16)]
            return carry

        lax.fori_loop(0, CH, edge, 0)

    def fire_s(i, row, sem):
        pltpu.async_copy(row, tbl.at[dst_all.at[i]], sem, add=True)

    def wait_s(i, row, sem):
        pltpu.make_async_copy(row, tbl.at[dst_all.at[i]], sem).wait()

    fire_g(0, saa, ada, ha, sga)

    def phase(i, sax, adx, hx, rowx, semgx, semsx,
              say, ady, hy, rowy, semgy, semsy, first):
        wait_g(i, sax, adx, hx, semgx)
        if not first:
            wait_s(i - 1, rowy, semsy)
        more = jnp.asarray(i + 1 < CHUNKS)

        @pl.when(more)
        def _():
            fire_g(i + 1, say, ady, hy, semgy)

        compute(sax, adx, hx, rowx)
        fire_s(i, rowx, semsx)

    def body(j, carry):
        i0 = 2 * j
        phase(i0, saa, ada, ha, rowa, sga, ssa,
              sab, adb, hb, rowb, sgb, ssb, False)
        phase(i0 + 1, sab, adb, hb, rowb, sgb, ssb,
              saa, ada, ha, rowa, sga, ssa, False)
        return carry

    phase(0, saa, ada, ha, rowa, sga, ssa,
          sab, adb, hb, rowb, sgb, ssb, True)
    phase(1, sab, adb, hb, rowb, sgb, ssb,
          saa, ada, ha, rowa, sga, ssa, False)
    lax.fori_loop(1, CHUNKS // 2, body, 0)
    wait_s(CHUNKS - 1, rowb, ssb)
    plsc.subcore_barrier()
    pltpu.sync_copy(tbl.at[pl.ds(r0, RPT)], out_hbm.at[cid, pl.ds(r0, RPT)])


@functools.partial(
    pl.kernel,
    out_type=jax.ShapeDtypeStruct((NC, NROWS, F), jnp.float32),
    mesh=_mesh,
    compiler_params=_sc_params,
    scratch_types=[
        pltpu.VMEM((CHUNKS, CH), jnp.int32),
        pltpu.VMEM((CHUNKS, CH), jnp.int32),
        pltpu.VMEM((CH, F), jnp.float32),
        pltpu.VMEM((CH, F), jnp.float32),
        pltpu.VMEM((CH, F), jnp.float32),
        pltpu.VMEM((CH, F), jnp.float32),
        pltpu.VMEM_SHARED((NROWS, F), jnp.float32),
        pltpu.SemaphoreType.DMA,
        pltpu.SemaphoreType.DMA,
        pltpu.SemaphoreType.DMA,
        pltpu.SemaphoreType.DMA,
        pltpu.SemaphoreType.DMA,
        pltpu.SemaphoreType.DMA,
        pltpu.SemaphoreType.DMA,
        pltpu.SemaphoreType.DMA,
    ],
)
def _sc2(src_hbm, dst_hbm, x32_hbm, zeros_hbm, out_hbm,
         src_all, dst_all, g0, g1, g2, g3, tbl,
         sg0, sg1, sg2, sg3, ss0, ss1, ss2, ss3):
    cid = lax.axis_index("c")
    sid = lax.axis_index("s")
    wid = cid * NS + sid
    r0 = pl.multiple_of(sid * RPT, 8)
    pltpu.sync_copy(zeros_hbm.at[pl.ds(r0, RPT)], tbl.at[pl.ds(r0, RPT)])
    pltpu.sync_copy(src_hbm.at[wid], src_all)
    pltpu.sync_copy(dst_hbm.at[wid], dst_all)
    plsc.subcore_barrier()

    gbufs = (g0, g1, g2, g3)
    gsems = (sg0, sg1, sg2, sg3)
    ssems = (ss0, ss1, ss2, ss3)

    def fire_g(i, b, sem):
        pltpu.async_copy(x32_hbm.at[src_all.at[i]], b, sem)

    def wait_g(i, b, sem):
        pltpu.make_async_copy(x32_hbm.at[src_all.at[i]], b, sem).wait()

    def fire_s(i, b, sem):
        pltpu.async_copy(b, tbl.at[dst_all.at[i]], sem, add=True)

    def wait_s(i, b, sem):
        pltpu.make_async_copy(b, tbl.at[dst_all.at[i]], sem).wait()

    for b in range(4):
        fire_g(b, gbufs[b], gsems[b])

    def body(j, carry):
        i0 = 4 * j
        for b in range(4):
            i = i0 + b
            wait_g(i, gbufs[b], gsems[b])
            fire_s(i, gbufs[b], ssems[b])
        for b in range(4):
            i = i0 + b

            @pl.when(jnp.asarray(i + 4 < CHUNKS))
            def _():
                wait_s(i, gbufs[b], ssems[b])
                fire_g(i + 4, gbufs[b], gsems[b])

        return carry

    lax.fori_loop(0, CHUNKS // 4, body, 0)
    for b in range(4):
        wait_s(CHUNKS - 4 + b, gbufs[b], ssems[b])
    plsc.subcore_barrier()
    pltpu.sync_copy(tbl.at[pl.ds(r0, RPT)], out_hbm.at[cid, pl.ds(r0, RPT)])


def _tca_body(x_ref, wg_ref, as_ref, ad_ref, k_ref,
              h_ref, as16_ref, ad16_ref, ssl16_ref, usl_ref):
    xb = x_ref[...]
    h = jnp.dot(xb, wg_ref[...], preferred_element_type=jnp.float32)
    a_s = jnp.dot(h, as_ref[...], preferred_element_type=jnp.float32)
    a_d = jnp.dot(h, ad_ref[...], preferred_element_type=jnp.float32)
    e = a_s + a_d
    s8 = jnp.exp(jnp.maximum(e, 0.2 * e))
    z8 = jnp.zeros((BN, HEADS), jnp.float32)
    h_ref[...] = h
    as16_ref[...] = jnp.concatenate([a_s, z8], axis=-1)
    ad16_ref[...] = jnp.concatenate([a_d, z8], axis=-1)
    ssl16_ref[...] = jnp.concatenate([s8, z8], axis=-1)
    sx64 = jnp.dot(s8, k_ref[...], preferred_element_type=jnp.float32)
    usl_ref[...] = h * sx64


def _tcb_body(p_ref, ssl16_ref, usl_ref, k_ref, bg_ref, x32_ref, inv_ref):
    den8 = (p_ref[0, :, F:F + HEADS] + p_ref[1, :, F:F + HEADS]
            + ssl16_ref[:, 0:HEADS])
    den = jnp.dot(den8, k_ref[...], preferred_element_type=jnp.float32)
    u = p_ref[0, :, 0:F] + p_ref[1, :, 0:F] + usl_ref[...]
    x32_ref[...] = jnp.maximum(u / den + bg_ref[...], 0.0)
    deg = p_ref[0, :, 72:73] + p_ref[1, :, 72:73]
    inv_ref[...] = jnp.broadcast_to(1.0 / jnp.maximum(deg, 1.0), (BN, F))


def _tcc_body(aggp_ref, inv_ref, x32_ref, wlt_ref, wrt_ref, bs_ref, out_ref):
    mean = (aggp_ref[0] + aggp_ref[1]) * inv_ref[...]
    out_ref[...] = (
        jnp.dot(mean, wlt_ref[...], preferred_element_type=jnp.float32)
        + jnp.dot(x32_ref[...], wrt_ref[...], preferred_element_type=jnp.float32)
        + bs_ref[...])


def kernel(x, edge_index, W1, b1, Wg, att_src, att_dst, bg, Wl, Wr, bs):
    idx = jnp.arange(F)
    a_s_m = jnp.zeros((F, HEADS), jnp.float32).at[
        idx, idx // HID].set(att_src.reshape(-1))
    a_d_m = jnp.zeros((F, HEADS), jnp.float32).at[
        idx, idx // HID].set(att_dst.reshape(-1))
    k_exp = jnp.repeat(jnp.eye(HEADS, dtype=jnp.float32), HID, axis=1)

    pad_j = jnp.arange(E_PAD - E, dtype=edge_index.dtype)
    src = jnp.concatenate([edge_index[0], pad_j % N])
    dst = jnp.concatenate([edge_index[1], N + 1 + pad_j % (NROWS - N - 1)])
    src3 = src.astype(jnp.int32).reshape(NW, CHUNKS, CH)
    dst3 = dst.astype(jnp.int32).reshape(NW, CHUNKS, CH)
    z80 = jnp.zeros((NROWS, ROWW), jnp.float32)
    z64 = jnp.zeros((NROWS, F), jnp.float32)

    h, as16, ad16, ssl16, u_sl = pl.pallas_call(
        _tca_body,
        grid=(GN,),
        in_specs=[
            pl.BlockSpec((BN, IN), lambda i: (i, 0)),
            pl.BlockSpec((IN, F), lambda i: (0, 0)),
            pl.BlockSpec((F, HEADS), lambda i: (0, 0)),
            pl.BlockSpec((F, HEADS), lambda i: (0, 0)),
            pl.BlockSpec((HEADS, F), lambda i: (0, 0)),
        ],
        out_specs=[
            pl.BlockSpec((BN, F), lambda i: (i, 0)),
            pl.BlockSpec((BN, 16), lambda i: (i, 0)),
            pl.BlockSpec((BN, 16), lambda i: (i, 0)),
            pl.BlockSpec((BN, 16), lambda i: (i, 0)),
            pl.BlockSpec((BN, F), lambda i: (i, 0)),
        ],
        out_shape=[
            jax.ShapeDtypeStruct((N, F), jnp.float32),
            jax.ShapeDtypeStruct((N, 16), jnp.float32),
            jax.ShapeDtypeStruct((N, 16), jnp.float32),
            jax.ShapeDtypeStruct((N, 16), jnp.float32),
            jax.ShapeDtypeStruct((N, F), jnp.float32),
        ],
    )(x, Wg, a_s_m, a_d_m, k_exp)

    ad16_p = jnp.concatenate(
        [ad16, jnp.zeros((NROWS - N, 16), jnp.float32)])

    part1 = _sc1(src3, dst3, as16, ad16_p, h, z80)

    x32, invd = pl.pallas_call(
        _tcb_body,
        grid=(GN,),
        in_specs=[
            pl.BlockSpec((NC, BN, ROWW), lambda i: (0, i, 0)),
            pl.BlockSpec((BN, 16), lambda i: (i, 0)),
            pl.BlockSpec((BN, F), lambda i: (i, 0)),
            pl.BlockSpec((HEADS, F), lambda i: (0, 0)),
            pl.BlockSpec((1, F), lambda i: (0, 0)),
        ],
        out_specs=[pl.BlockSpec((BN, F), lambda i: (i, 0))] * 2,
        out_shape=[jax.ShapeDtypeStruct((N, F), jnp.float32)] * 2,
    )(part1, ssl16, u_sl, k_exp, bg.reshape(1, F))

    part2 = _sc2(src3, dst3, x32, z64)

    out = pl.pallas_call(
        _tcc_body,
        grid=(GN,),
        in_specs=[
            pl.BlockSpec((NC, BN, F), lambda i: (0, i, 0)),
            pl.BlockSpec((BN, F), lambda i: (i, 0)),
            pl.BlockSpec((BN, F), lambda i: (i, 0)),
            pl.BlockSpec((F, F), lambda i: (0, 0)),
            pl.BlockSpec((F, F), lambda i: (0, 0)),
            pl.BlockSpec((1, F), lambda i: (0, 0)),
        ],
        out_specs=pl.BlockSpec((BN, F), lambda i: (i, 0)),
        out_shape=jax.ShapeDtypeStruct((N, F), jnp.float32),
    )(part2, invd, x32, Wl.T, Wr.T, bs.reshape(1, F))

    return (x32, out)


# speedup vs baseline: 75.7067x; 1.0059x over previous
import functools

import jax
import jax.numpy as jnp
from jax import lax
from jax.experimental import pallas as pl
from jax.experimental.pallas import tpu as pltpu
from jax.experimental.pallas import tpu_sc as plsc

N = 10000
E = 320000
IN = 128
HID = 8
HEADS = 8
F = 64
ROWW = 80

NC = 2
NS = 16
NW = NC * NS
CH = 128
CHUNKS = 80
E_PAD = NW * CHUNKS * CH
RPT = 8 * (-(-(N + 1) // (NS * 8)))
NROWS = RPT * NS
BN = 400
GN = N // BN

_mesh = plsc.VectorSubcoreMesh(
    core_axis_name="c", subcore_axis_name="s", num_cores=NC, num_subcores=NS)
_sc_params = pltpu.CompilerParams(use_tc_tiling_on_sc=False)


@functools.partial(
    pl.kernel,
    out_type=jax.ShapeDtypeStruct((NC, NROWS, ROWW), jnp.float32),
    mesh=_mesh,
    compiler_params=_sc_params,
    scratch_types=[
        pltpu.VMEM((CHUNKS, CH), jnp.int32),
        pltpu.VMEM((CHUNKS, CH), jnp.int32),
        pltpu.VMEM((CH, 16), jnp.float32),
        pltpu.VMEM((CH, 16), jnp.float32),
        pltpu.VMEM((CH, 16), jnp.float32),
        pltpu.VMEM((CH, 16), jnp.float32),
        pltpu.VMEM((CH, F), jnp.float32),
        pltpu.VMEM((CH, F), jnp.float32),
        pltpu.VMEM((CH, ROWW), jnp.float32),
        pltpu.VMEM((CH, ROWW), jnp.float32),
        pltpu.VMEM_SHARED((NROWS, ROWW), jnp.float32),
        pltpu.SemaphoreType.DMA,
        pltpu.SemaphoreType.DMA,
        pltpu.SemaphoreType.DMA,
        pltpu.SemaphoreType.DMA,
    ],
)
def _sc1(src_hbm, dst_hbm, as16_hbm, ad16_hbm, h_hbm, zeros_hbm, out_hbm,
         src_all, dst_all, saa, sab, ada, adb, ha, hb, rowa, rowb, tbl,
         sga, sgb, ssa, ssb):
    cid = lax.axis_index("c")
    sid = lax.axis_index("s")
    wid = cid * NS + sid
    r0 = pl.multiple_of(sid * RPT, 8)
    pltpu.sync_copy(zeros_hbm.at[pl.ds(r0, RPT)], tbl.at[pl.ds(r0, RPT)])
    pltpu.sync_copy(src_hbm.at[wid], src_all)
    pltpu.sync_copy(dst_hbm.at[wid], dst_all)
    plsc.subcore_barrier()

    def fire_g(i, sa, ad, h, sem):
        pltpu.async_copy(as16_hbm.at[src_all.at[i]], sa, sem)
        pltpu.async_copy(ad16_hbm.at[dst_all.at[i]], ad, sem)
        pltpu.async_copy(h_hbm.at[src_all.at[i]], h, sem)

    def wait_g(i, sa, ad, h, sem):
        pltpu.make_async_copy(as16_hbm.at[src_all.at[i]], sa, sem).wait()
        pltpu.make_async_copy(ad16_hbm.at[dst_all.at[i]], ad, sem).wait()
        pltpu.make_async_copy(h_hbm.at[src_all.at[i]], h, sem).wait()

    lane = lax.iota(jnp.int32, 16)
    perm_idx = [jnp.where(lane < 8, 2 * k, 2 * k + 1) for k in range(4)]
    _dnums = lax.GatherDimensionNumbers(
        offset_dims=(), collapsed_slice_dims=(0,), start_index_map=(0,))

    def dyn_gather(v, idxv):
        return lax.gather(
            v, idxv[:, None], _dnums, slice_sizes=(1,),
            mode=lax.GatherScatterMode.PROMISE_IN_BOUNDS)

    def compute(sa, ad, h, row):
        def edge4(r4, carry):
            for rr in range(4):
                r = 4 * r4 + rr
                a = sa[r, pl.ds(0, 16)] + ad[r, pl.ds(0, 16)]
                s = jnp.exp(jnp.maximum(a, 0.2 * a))
                row[r, pl.ds(F, 16)] = s
                for k in range(4):
                    m = dyn_gather(s, perm_idx[k])
                    row[r, pl.ds(16 * k, 16)] = m * h[r, pl.ds(16 * k, 16)]
            return carry

        lax.fori_loop(0, CH // 4, edge4, 0)

    def fire_s(i, row, sem):
        pltpu.async_copy(row, tbl.at[dst_all.at[i]], sem, add=True)

    def wait_s(i, row, sem):
        pltpu.make_async_copy(row, tbl.at[dst_all.at[i]], sem).wait()

    fire_g(0, saa, ada, ha, sga)

    def phase(i, sax, adx, hx, rowx, semgx, semsx,
              say, ady, hy, rowy, semgy, semsy, first):
        wait_g(i, sax, adx, hx, semgx)
        if not first:
            wait_s(i - 1, rowy, semsy)
        more = jnp.asarray(i + 1 < CHUNKS)

        @pl.when(more)
        def _():
            fire_g(i + 1, say, ady, hy, semgy)

        compute(sax, adx, hx, rowx)
        fire_s(i, rowx, semsx)

    def body(j, carry):
        i0 = 2 * j
        phase(i0, saa, ada, ha, rowa, sga, ssa,
              sab, adb, hb, rowb, sgb, ssb, False)
        phase(i0 + 1, sab, adb, hb, rowb, sgb, ssb,
              saa, ada, ha, rowa, sga, ssa, False)
        return carry

    phase(0, saa, ada, ha, rowa, sga, ssa,
          sab, adb, hb, rowb, sgb, ssb, True)
    phase(1, sab, adb, hb, rowb, sgb, ssb,
          saa, ada, ha, rowa, sga, ssa, False)
    lax.fori_loop(1, CHUNKS // 2, body, 0)
    wait_s(CHUNKS - 1, rowb, ssb)
    plsc.subcore_barrier()
    pltpu.sync_copy(tbl.at[pl.ds(r0, RPT)], out_hbm.at[cid, pl.ds(r0, RPT)])


@functools.partial(
    pl.kernel,
    out_type=jax.ShapeDtypeStruct((NC, NROWS, F), jnp.float32),
    mesh=_mesh,
    compiler_params=_sc_params,
    scratch_types=[
        pltpu.VMEM((CHUNKS, CH), jnp.int32),
        pltpu.VMEM((CHUNKS, CH), jnp.int32),
        pltpu.VMEM((CH, F), jnp.float32),
        pltpu.VMEM((CH, F), jnp.float32),
        pltpu.VMEM((CH, F), jnp.float32),
        pltpu.VMEM((CH, F), jnp.float32),
        pltpu.VMEM_SHARED((NROWS, F), jnp.float32),
        pltpu.SemaphoreType.DMA,
        pltpu.SemaphoreType.DMA,
        pltpu.SemaphoreType.DMA,
        pltpu.SemaphoreType.DMA,
        pltpu.SemaphoreType.DMA,
        pltpu.SemaphoreType.DMA,
        pltpu.SemaphoreType.DMA,
        pltpu.SemaphoreType.DMA,
    ],
)
def _sc2(src_hbm, dst_hbm, x32_hbm, zeros_hbm, out_hbm,
         src_all, dst_all, g0, g1, g2, g3, tbl,
         sg0, sg1, sg2, sg3, ss0, ss1, ss2, ss3):
    cid = lax.axis_index("c")
    sid = lax.axis_index("s")
    wid = cid * NS + sid
    r0 = pl.multiple_of(sid * RPT, 8)
    pltpu.sync_copy(zeros_hbm.at[pl.ds(r0, RPT)], tbl.at[pl.ds(r0, RPT)])
    pltpu.sync_copy(src_hbm.at[wid], src_all)
    pltpu.sync_copy(dst_hbm.at[wid], dst_all)
    plsc.subcore_barrier()

    gbufs = (g0, g1, g2, g3)
    gsems = (sg0, sg1, sg2, sg3)
    ssems = (ss0, ss1, ss2, ss3)

    def fire_g(i, b, sem):
        pltpu.async_copy(x32_hbm.at[src_all.at[i]], b, sem)

    def wait_g(i, b, sem):
        pltpu.make_async_copy(x32_hbm.at[src_all.at[i]], b, sem).wait()

    def fire_s(i, b, sem):
        pltpu.async_copy(b, tbl.at[dst_all.at[i]], sem, add=True)

    def wait_s(i, b, sem):
        pltpu.make_async_copy(b, tbl.at[dst_all.at[i]], sem).wait()

    for b in range(4):
        fire_g(b, gbufs[b], gsems[b])

    def body(j, carry):
        i0 = 4 * j
        for b in range(4):
            i = i0 + b
            wait_g(i, gbufs[b], gsems[b])
            fire_s(i, gbufs[b], ssems[b])
        for b in range(4):
            i = i0 + b

            @pl.when(jnp.asarray(i + 4 < CHUNKS))
            def _():
                wait_s(i, gbufs[b], ssems[b])
                fire_g(i + 4, gbufs[b], gsems[b])

        return carry

    lax.fori_loop(0, CHUNKS // 4, body, 0)
    for b in range(4):
        wait_s(CHUNKS - 4 + b, gbufs[b], ssems[b])
    plsc.subcore_barrier()
    pltpu.sync_copy(tbl.at[pl.ds(r0, RPT)], out_hbm.at[cid, pl.ds(r0, RPT)])


def _tca_body(x_ref, wg_ref, as_ref, ad_ref, k_ref,
              h_ref, as16_ref, ad16_ref, ssl16_ref, usl_ref):
    xb = x_ref[...]
    h = jnp.dot(xb, wg_ref[...], preferred_element_type=jnp.float32)
    a_s = jnp.dot(h, as_ref[...], preferred_element_type=jnp.float32)
    a_d = jnp.dot(h, ad_ref[...], preferred_element_type=jnp.float32)
    e = a_s + a_d
    s8 = jnp.exp(jnp.maximum(e, 0.2 * e))
    z8 = jnp.zeros((BN, HEADS), jnp.float32)
    h_ref[...] = h
    as16_ref[...] = jnp.concatenate([a_s, z8], axis=-1)
    ad16_ref[...] = jnp.concatenate([a_d, z8], axis=-1)
    ssl16_ref[...] = jnp.concatenate([s8, z8], axis=-1)
    sx64 = jnp.dot(s8, k_ref[...], preferred_element_type=jnp.float32)
    usl_ref[...] = h * sx64


def _tcb_body(p_ref, ssl16_ref, usl_ref, k_ref, bg_ref, x32_ref, inv_ref):
    den8 = (p_ref[0, :, F:F + HEADS] + p_ref[1, :, F:F + HEADS]
            + ssl16_ref[:, 0:HEADS])
    den = jnp.dot(den8, k_ref[...], preferred_element_type=jnp.float32)
    u = p_ref[0, :, 0:F] + p_ref[1, :, 0:F] + usl_ref[...]
    x32_ref[...] = jnp.maximum(u / den + bg_ref[...], 0.0)
    deg = p_ref[0, :, 72:73] + p_ref[1, :, 72:73]
    inv_ref[...] = jnp.broadcast_to(1.0 / jnp.maximum(deg, 1.0), (BN, F))


def _tcc_body(aggp_ref, inv_ref, x32_ref, wlt_ref, wrt_ref, bs_ref, out_ref):
    mean = (aggp_ref[0] + aggp_ref[1]) * inv_ref[...]
    out_ref[...] = (
        jnp.dot(mean, wlt_ref[...], preferred_element_type=jnp.float32)
        + jnp.dot(x32_ref[...], wrt_ref[...], preferred_element_type=jnp.float32)
        + bs_ref[...])


def kernel(x, edge_index, W1, b1, Wg, att_src, att_dst, bg, Wl, Wr, bs):
    idx = jnp.arange(F)
    a_s_m = jnp.zeros((F, HEADS), jnp.float32).at[
        idx, idx // HID].set(att_src.reshape(-1))
    a_d_m = jnp.zeros((F, HEADS), jnp.float32).at[
        idx, idx // HID].set(att_dst.reshape(-1))
    k_exp = jnp.repeat(jnp.eye(HEADS, dtype=jnp.float32), HID, axis=1)

    pad_j = jnp.arange(E_PAD - E, dtype=edge_index.dtype)
    src = jnp.concatenate([edge_index[0], pad_j % N])
    dst = jnp.concatenate([edge_index[1], N + 1 + pad_j % (NROWS - N - 1)])
    src3 = src.astype(jnp.int32).reshape(NW, CHUNKS, CH)
    dst3 = dst.astype(jnp.int32).reshape(NW, CHUNKS, CH)
    z80 = jnp.zeros((NROWS, ROWW), jnp.float32)
    z64 = jnp.zeros((NROWS, F), jnp.float32)

    h, as16, ad16, ssl16, u_sl = pl.pallas_call(
        _tca_body,
        grid=(GN,),
        in_specs=[
            pl.BlockSpec((BN, IN), lambda i: (i, 0)),
            pl.BlockSpec((IN, F), lambda i: (0, 0)),
            pl.BlockSpec((F, HEADS), lambda i: (0, 0)),
            pl.BlockSpec((F, HEADS), lambda i: (0, 0)),
            pl.BlockSpec((HEADS, F), lambda i: (0, 0)),
        ],
        out_specs=[
            pl.BlockSpec((BN, F), lambda i: (i, 0)),
            pl.BlockSpec((BN, 16), lambda i: (i, 0)),
            pl.BlockSpec((BN, 16), lambda i: (i, 0)),
            pl.BlockSpec((BN, 16), lambda i: (i, 0)),
            pl.BlockSpec((BN, F), lambda i: (i, 0)),
        ],
        out_shape=[
            jax.ShapeDtypeStruct((N, F), jnp.float32),
            jax.ShapeDtypeStruct((N, 16), jnp.float32),
            jax.ShapeDtypeStruct((N, 16), jnp.float32),
            jax.ShapeDtypeStruct((N, 16), jnp.float32),
            jax.ShapeDtypeStruct((N, F), jnp.float32),
        ],
    )(x, Wg, a_s_m, a_d_m, k_exp)

    ad16_p = jnp.concatenate(
        [ad16, jnp.zeros((NROWS - N, 16), jnp.float32)])

    part1 = _sc1(src3, dst3, as16, ad16_p, h, z80)

    x32, invd = pl.pallas_call(
        _tcb_body,
        grid=(GN,),
        in_specs=[
            pl.BlockSpec((NC, BN, ROWW), lambda i: (0, i, 0)),
            pl.BlockSpec((BN, 16), lambda i: (i, 0)),
            pl.BlockSpec((BN, F), lambda i: (i, 0)),
            pl.BlockSpec((HEADS, F), lambda i: (0, 0)),
            pl.BlockSpec((1, F), lambda i: (0, 0)),
        ],
        out_specs=[pl.BlockSpec((BN, F), lambda i: (i, 0))] * 2,
        out_shape=[jax.ShapeDtypeStruct((N, F), jnp.float32)] * 2,
    )(part1, ssl16, u_sl, k_exp, bg.reshape(1, F))

    part2 = _sc2(src3, dst3, x32, z64)

    out = pl.pallas_call(
        _tcc_body,
        grid=(GN,),
        in_specs=[
            pl.BlockSpec((NC, BN, F), lambda i: (0, i, 0)),
            pl.BlockSpec((BN, F), lambda i: (i, 0)),
            pl.BlockSpec((BN, F), lambda i: (i, 0)),
            pl.BlockSpec((F, F), lambda i: (0, 0)),
            pl.BlockSpec((F, F), lambda i: (0, 0)),
            pl.BlockSpec((1, F), lambda i: (0, 0)),
        ],
        out_specs=pl.BlockSpec((BN, F), lambda i: (i, 0)),
        out_shape=jax.ShapeDtypeStruct((N, F), jnp.float32),
    )(part2, invd, x32, Wl.T, Wr.T, bs.reshape(1, F))

    return (x32, out)


# speedup vs baseline: 80.1046x; 1.0581x over previous
import functools

import jax
import jax.numpy as jnp
from jax import lax
from jax.experimental import pallas as pl
from jax.experimental.pallas import tpu as pltpu
from jax.experimental.pallas import tpu_sc as plsc

N = 10000
E = 320000
IN = 128
HID = 8
HEADS = 8
F = 64
ROWW = 80

NC = 2
NS = 16
NW = NC * NS
CH = 128
CHUNKS = 80
E_PAD = NW * CHUNKS * CH
RPT = 8 * (-(-(N + 1) // (NS * 8)))
NROWS = RPT * NS
BN = 400
GN = N // BN

_mesh = plsc.VectorSubcoreMesh(
    core_axis_name="c", subcore_axis_name="s", num_cores=NC, num_subcores=NS)
_sc_params = pltpu.CompilerParams(use_tc_tiling_on_sc=False)


@functools.partial(
    pl.kernel,
    out_type=jax.ShapeDtypeStruct((NC, NROWS, ROWW), jnp.float32),
    mesh=_mesh,
    compiler_params=_sc_params,
    scratch_types=[
        pltpu.VMEM((CHUNKS, CH), jnp.int32),
        pltpu.VMEM((CHUNKS, CH), jnp.int32),
        pltpu.VMEM((CH, 16), jnp.float32),
        pltpu.VMEM((CH, 16), jnp.float32),
        pltpu.VMEM((CH, 16), jnp.float32),
        pltpu.VMEM((CH, 16), jnp.float32),
        pltpu.VMEM((CH, F), jnp.float32),
        pltpu.VMEM((CH, F), jnp.float32),
        pltpu.VMEM((CH, ROWW), jnp.float32),
        pltpu.VMEM((CH, ROWW), jnp.float32),
        pltpu.VMEM_SHARED((NROWS, ROWW), jnp.float32),
        pltpu.SemaphoreType.DMA,
        pltpu.SemaphoreType.DMA,
        pltpu.SemaphoreType.DMA,
        pltpu.SemaphoreType.DMA,
    ],
)
def _sc1(src_hbm, dst_hbm, as16_hbm, ad16_hbm, h_hbm, zeros_hbm, out_hbm,
         src_all, dst_all, saa, sab, ada, adb, ha, hb, rowa, rowb, tbl,
         sga, sgb, ssa, ssb):
    cid = lax.axis_index("c")
    sid = lax.axis_index("s")
    wid = cid * NS + sid
    r0 = pl.multiple_of(sid * RPT, 8)
    pltpu.sync_copy(zeros_hbm.at[pl.ds(r0, RPT)], tbl.at[pl.ds(r0, RPT)])
    pltpu.sync_copy(src_hbm.at[wid], src_all)
    pltpu.sync_copy(dst_hbm.at[wid], dst_all)
    plsc.subcore_barrier()

    def fire_g(i, sa, ad, h, sem):
        pltpu.async_copy(as16_hbm.at[src_all.at[i]], sa, sem)
        pltpu.async_copy(ad16_hbm.at[dst_all.at[i]], ad, sem)
        pltpu.async_copy(h_hbm.at[src_all.at[i]], h, sem)

    def wait_g(i, sa, ad, h, sem):
        pltpu.make_async_copy(as16_hbm.at[src_all.at[i]], sa, sem).wait()
        pltpu.make_async_copy(ad16_hbm.at[dst_all.at[i]], ad, sem).wait()
        pltpu.make_async_copy(h_hbm.at[src_all.at[i]], h, sem).wait()

    lane = lax.iota(jnp.int32, 16)
    perm_idx = [jnp.where(lane < 8, 2 * k, 2 * k + 1) for k in range(4)]
    _dnums = lax.GatherDimensionNumbers(
        offset_dims=(), collapsed_slice_dims=(0,), start_index_map=(0,))

    def dyn_gather(v, idxv):
        return lax.gather(
            v, idxv[:, None], _dnums, slice_sizes=(1,),
            mode=lax.GatherScatterMode.PROMISE_IN_BOUNDS)

    def compute(sa, ad, h, row):
        def edge4(r4, carry):
            for rr in range(4):
                r = 4 * r4 + rr
                a = sa[r, pl.ds(0, 16)] + ad[r, pl.ds(0, 16)]
                s = jnp.exp(jnp.maximum(a, 0.2 * a))
                row[r, pl.ds(F, 16)] = s
                for k in range(4):
                    m = dyn_gather(s, perm_idx[k])
                    row[r, pl.ds(16 * k, 16)] = m * h[r, pl.ds(16 * k, 16)]
            return carry

        lax.fori_loop(0, CH // 4, edge4, 0)

    def fire_s(i, row, sem):
        pltpu.async_copy(row, tbl.at[dst_all.at[i]], sem, add=True)

    def wait_s(i, row, sem):
        pltpu.make_async_copy(row, tbl.at[dst_all.at[i]], sem).wait()

    fire_g(0, saa, ada, ha, sga)

    def phase(i, sax, adx, hx, rowx, semgx, semsx,
              say, ady, hy, rowy, semgy, semsy, first):
        wait_g(i, sax, adx, hx, semgx)
        more = jnp.asarray(i + 1 < CHUNKS)

        @pl.when(more)
        def _():
            fire_g(i + 1, say, ady, hy, semgy)

        compute(sax, adx, hx, rowx)
        if not first:
            wait_s(i - 1, rowy, semsy)
        fire_s(i, rowx, semsx)

    def body(j, carry):
        i0 = 2 * j
        phase(i0, saa, ada, ha, rowa, sga, ssa,
              sab, adb, hb, rowb, sgb, ssb, False)
        phase(i0 + 1, sab, adb, hb, rowb, sgb, ssb,
              saa, ada, ha, rowa, sga, ssa, False)
        return carry

    phase(0, saa, ada, ha, rowa, sga, ssa,
          sab, adb, hb, rowb, sgb, ssb, True)
    phase(1, sab, adb, hb, rowb, sgb, ssb,
          saa, ada, ha, rowa, sga, ssa, False)
    lax.fori_loop(1, CHUNKS // 2, body, 0)
    wait_s(CHUNKS - 1, rowb, ssb)
    plsc.subcore_barrier()
    pltpu.sync_copy(tbl.at[pl.ds(r0, RPT)], out_hbm.at[cid, pl.ds(r0, RPT)])


@functools.partial(
    pl.kernel,
    out_type=jax.ShapeDtypeStruct((NC, NROWS, F), jnp.float32),
    mesh=_mesh,
    compiler_params=_sc_params,
    scratch_types=[
        pltpu.VMEM((CHUNKS, CH), jnp.int32),
        pltpu.VMEM((CHUNKS, CH), jnp.int32),
        pltpu.VMEM((CH, F), jnp.float32),
        pltpu.VMEM((CH, F), jnp.float32),
        pltpu.VMEM((CH, F), jnp.float32),
        pltpu.VMEM((CH, F), jnp.float32),
        pltpu.VMEM_SHARED((NROWS, F), jnp.float32),
        pltpu.SemaphoreType.DMA,
        pltpu.SemaphoreType.DMA,
        pltpu.SemaphoreType.DMA,
        pltpu.SemaphoreType.DMA,
        pltpu.SemaphoreType.DMA,
        pltpu.SemaphoreType.DMA,
        pltpu.SemaphoreType.DMA,
        pltpu.SemaphoreType.DMA,
    ],
)
def _sc2(src_hbm, dst_hbm, x32_hbm, zeros_hbm, out_hbm,
         src_all, dst_all, g0, g1, g2, g3, tbl,
         sg0, sg1, sg2, sg3, ss0, ss1, ss2, ss3):
    cid = lax.axis_index("c")
    sid = lax.axis_index("s")
    wid = cid * NS + sid
    r0 = pl.multiple_of(sid * RPT, 8)
    pltpu.sync_copy(zeros_hbm.at[pl.ds(r0, RPT)], tbl.at[pl.ds(r0, RPT)])
    pltpu.sync_copy(src_hbm.at[wid], src_all)
    pltpu.sync_copy(dst_hbm.at[wid], dst_all)
    plsc.subcore_barrier()

    gbufs = (g0, g1, g2, g3)
    gsems = (sg0, sg1, sg2, sg3)
    ssems = (ss0, ss1, ss2, ss3)

    def fire_g(i, b, sem):
        pltpu.async_copy(x32_hbm.at[src_all.at[i]], b, sem)

    def wait_g(i, b, sem):
        pltpu.make_async_copy(x32_hbm.at[src_all.at[i]], b, sem).wait()

    def fire_s(i, b, sem):
        pltpu.async_copy(b, tbl.at[dst_all.at[i]], sem, add=True)

    def wait_s(i, b, sem):
        pltpu.make_async_copy(b, tbl.at[dst_all.at[i]], sem).wait()

    for b in range(4):
        fire_g(b, gbufs[b], gsems[b])

    def body(j, carry):
        i0 = 4 * j
        for b in range(4):
            i = i0 + b
            wait_g(i, gbufs[b], gsems[b])
            fire_s(i, gbufs[b], ssems[b])
        for b in range(4):
            i = i0 + b

            @pl.when(jnp.asarray(i + 4 < CHUNKS))
            def _():
                wait_s(i, gbufs[b], ssems[b])
                fire_g(i + 4, gbufs[b], gsems[b])

        return carry

    lax.fori_loop(0, CHUNKS // 4, body, 0)
    for b in range(4):
        wait_s(CHUNKS - 4 + b, gbufs[b], ssems[b])
    plsc.subcore_barrier()
    pltpu.sync_copy(tbl.at[pl.ds(r0, RPT)], out_hbm.at[cid, pl.ds(r0, RPT)])


def _tca_body(x_ref, wg_ref, as_ref, ad_ref, k_ref,
              h_ref, as16_ref, ad16_ref, ssl16_ref, usl_ref):
    xb = x_ref[...]
    h = jnp.dot(xb, wg_ref[...], preferred_element_type=jnp.float32)
    a_s = jnp.dot(h, as_ref[...], preferred_element_type=jnp.float32)
    a_d = jnp.dot(h, ad_ref[...], preferred_element_type=jnp.float32)
    e = a_s + a_d
    s8 = jnp.exp(jnp.maximum(e, 0.2 * e))
    z8 = jnp.zeros((BN, HEADS), jnp.float32)
    h_ref[...] = h
    as16_ref[...] = jnp.concatenate([a_s, z8], axis=-1)
    ad16_ref[...] = jnp.concatenate([a_d, z8], axis=-1)
    ssl16_ref[...] = jnp.concatenate([s8, z8], axis=-1)
    sx64 = jnp.dot(s8, k_ref[...], preferred_element_type=jnp.float32)
    usl_ref[...] = h * sx64


def _tcb_body(p_ref, ssl16_ref, usl_ref, k_ref, bg_ref, x32_ref, inv_ref):
    den8 = (p_ref[0, :, F:F + HEADS] + p_ref[1, :, F:F + HEADS]
            + ssl16_ref[:, 0:HEADS])
    den = jnp.dot(den8, k_ref[...], preferred_element_type=jnp.float32)
    u = p_ref[0, :, 0:F] + p_ref[1, :, 0:F] + usl_ref[...]
    x32_ref[...] = jnp.maximum(u / den + bg_ref[...], 0.0)
    deg = p_ref[0, :, 72:73] + p_ref[1, :, 72:73]
    inv_ref[...] = jnp.broadcast_to(1.0 / jnp.maximum(deg, 1.0), (BN, F))


def _tcc_body(aggp_ref, inv_ref, x32_ref, wlt_ref, wrt_ref, bs_ref, out_ref):
    mean = (aggp_ref[0] + aggp_ref[1]) * inv_ref[...]
    out_ref[...] = (
        jnp.dot(mean, wlt_ref[...], preferred_element_type=jnp.float32)
        + jnp.dot(x32_ref[...], wrt_ref[...], preferred_element_type=jnp.float32)
        + bs_ref[...])


def kernel(x, edge_index, W1, b1, Wg, att_src, att_dst, bg, Wl, Wr, bs):
    idx = jnp.arange(F)
    a_s_m = jnp.zeros((F, HEADS), jnp.float32).at[
        idx, idx // HID].set(att_src.reshape(-1))
    a_d_m = jnp.zeros((F, HEADS), jnp.float32).at[
        idx, idx // HID].set(att_dst.reshape(-1))
    k_exp = jnp.repeat(jnp.eye(HEADS, dtype=jnp.float32), HID, axis=1)

    pad_j = jnp.arange(E_PAD - E, dtype=edge_index.dtype)
    src = jnp.concatenate([edge_index[0], pad_j % N])
    dst = jnp.concatenate([edge_index[1], N + 1 + pad_j % (NROWS - N - 1)])
    src3 = src.astype(jnp.int32).reshape(NW, CHUNKS, CH)
    dst3 = dst.astype(jnp.int32).reshape(NW, CHUNKS, CH)
    z80 = jnp.zeros((NROWS, ROWW), jnp.float32)
    z64 = jnp.zeros((NROWS, F), jnp.float32)

    h, as16, ad16, ssl16, u_sl = pl.pallas_call(
        _tca_body,
        grid=(GN,),
        in_specs=[
            pl.BlockSpec((BN, IN), lambda i: (i, 0)),
            pl.BlockSpec((IN, F), lambda i: (0, 0)),
            pl.BlockSpec((F, HEADS), lambda i: (0, 0)),
            pl.BlockSpec((F, HEADS), lambda i: (0, 0)),
            pl.BlockSpec((HEADS, F), lambda i: (0, 0)),
        ],
        out_specs=[
            pl.BlockSpec((BN, F), lambda i: (i, 0)),
            pl.BlockSpec((BN, 16), lambda i: (i, 0)),
            pl.BlockSpec((BN, 16), lambda i: (i, 0)),
            pl.BlockSpec((BN, 16), lambda i: (i, 0)),
            pl.BlockSpec((BN, F), lambda i: (i, 0)),
        ],
        out_shape=[
            jax.ShapeDtypeStruct((N, F), jnp.float32),
            jax.ShapeDtypeStruct((N, 16), jnp.float32),
            jax.ShapeDtypeStruct((N, 16), jnp.float32),
            jax.ShapeDtypeStruct((N, 16), jnp.float32),
            jax.ShapeDtypeStruct((N, F), jnp.float32),
        ],
    )(x, Wg, a_s_m, a_d_m, k_exp)

    ad16_p = jnp.concatenate(
        [ad16, jnp.zeros((NROWS - N, 16), jnp.float32)])

    part1 = _sc1(src3, dst3, as16, ad16_p, h, z80)

    x32, invd = pl.pallas_call(
        _tcb_body,
        grid=(GN,),
        in_specs=[
            pl.BlockSpec((NC, BN, ROWW), lambda i: (0, i, 0)),
            pl.BlockSpec((BN, 16), lambda i: (i, 0)),
            pl.BlockSpec((BN, F), lambda i: (i, 0)),
            pl.BlockSpec((HEADS, F), lambda i: (0, 0)),
            pl.BlockSpec((1, F), lambda i: (0, 0)),
        ],
        out_specs=[pl.BlockSpec((BN, F), lambda i: (i, 0))] * 2,
        out_shape=[jax.ShapeDtypeStruct((N, F), jnp.float32)] * 2,
    )(part1, ssl16, u_sl, k_exp, bg.reshape(1, F))

    part2 = _sc2(src3, dst3, x32, z64)

    out = pl.pallas_call(
        _tcc_body,
        grid=(GN,),
        in_specs=[
            pl.BlockSpec((NC, BN, F), lambda i: (0, i, 0)),
            pl.BlockSpec((BN, F), lambda i: (i, 0)),
            pl.BlockSpec((BN, F), lambda i: (i, 0)),
            pl.BlockSpec((F, F), lambda i: (0, 0)),
            pl.BlockSpec((F, F), lambda i: (0, 0)),
            pl.BlockSpec((1, F), lambda i: (0, 0)),
        ],
        out_specs=pl.BlockSpec((BN, F), lambda i: (i, 0)),
        out_shape=jax.ShapeDtypeStruct((N, F), jnp.float32),
    )(part2, invd, x32, Wl.T, Wr.T, bs.reshape(1, F))

    return (x32, out)
